# Initial kernel scaffold; baseline (speedup 1.0000x reference)
import jax
import jax.numpy as jnp
from jax.experimental import pallas as pl


def kernel(logits, targets, edge_batch, num_graphs):
    raise NotImplementedError("write your pallas kernel here")


# speedup vs baseline: 30.9690x; 30.9690x over previous
import functools

import jax
import jax.numpy as jnp
from jax import lax
from jax.experimental import pallas as pl
from jax.experimental.pallas import tpu as pltpu
from jax.experimental.pallas import tpu_sc as plsc

E = 800000
G = 64
NW = 32
CH = E // NW
CHP = CH + 8
K = 16
INV_TEMP = 20.0
MARGIN = 0.2
PAIR_W = 0.3
NEG_BIG = -1e30
NEG_INF = float("-inf")
TINY = 1.1754943508222875e-38

BR = 16
E_PAD = 800768
ROWS = E_PAD // 128
NB = ROWS // BR


_mesh = plsc.VectorSubcoreMesh(core_axis_name="c", subcore_axis_name="s")

NSTAT = 7
WSTAT = NSTAT * G * K


@functools.partial(
    pl.kernel,
    out_type=jax.ShapeDtypeStruct((NW * WSTAT,), jnp.float32),
    mesh=_mesh,
    compiler_params=pltpu.CompilerParams(needs_layout_passes=False),
    scratch_types=[
        pltpu.VMEM((CHP,), jnp.float32),
        pltpu.VMEM((CHP,), jnp.float32),
        pltpu.VMEM((CHP,), jnp.int32),
        pltpu.VMEM((80,), jnp.int32),
        pltpu.VMEM((WSTAT,), jnp.float32),
    ],
)
def _sc_part(logits_hbm, targets_hbm, eb_hbm, o_st, lg_v, tg_v, eb_v, lb_v, v_st):
    c = lax.axis_index("c")
    s = lax.axis_index("s")
    wid = s * 2 + c
    base = wid * CH
    pltpu.sync_copy(logits_hbm.at[pl.ds(base, CH)], lg_v.at[pl.ds(0, CH)])
    pltpu.sync_copy(targets_hbm.at[pl.ds(base, CH)], tg_v.at[pl.ds(0, CH)])
    pltpu.sync_copy(eb_hbm.at[pl.ds(base, CH)], eb_v.at[pl.ds(0, CH)])

    lane = lax.iota(jnp.int32, 16)

    for r in range(4):
        gvec = lane + r * 16

        def bs_body(i, carry):
            lo, hi = carry
            active = lo < hi
            mid = lax.div(lo + hi, 2)
            vals = plsc.load_gather(eb_v, [mid])
            right = vals < gvec
            lo2 = jnp.where(active & right, mid + 1, lo)
            hi2 = jnp.where(active & (~right), mid, hi)
            return lo2, hi2

        lo, hi = lax.fori_loop(
            0, 15, bs_body,
            (jnp.zeros((16,), jnp.int32), jnp.full((16,), CH, jnp.int32)))
        lb_v[pl.ds(r * 16, 16)] = lo
    lb_v[pl.ds(64, 16)] = jnp.full((16,), CH, jnp.int32)

    def graph_body(g, _):
        gfull = jnp.full((16,), g, jnp.int32)
        start_v = plsc.load_gather(lb_v, [gfull])
        end_v = plsc.load_gather(lb_v, [gfull + 1])
        start = lax.reduce_max(start_v, (0,))
        end = lax.reduce_max(end_v, (0,))
        i0 = lax.div(start, 16)
        i1 = lax.div(end + 15, 16)

        def elem_body(i, carry):
            m_a, s_a, m_p, s_p, pc, top = carry
            off = i * 16
            vraw = lg_v[pl.ds(off, 16)]
            trg = tg_v[pl.ds(off, 16)]
            gidx = off + lane
            valid = (gidx >= start) & (gidx < end)
            posm = valid & (trg > 0.5)
            sc_v = vraw * INV_TEMP
            sv = jnp.where(valid, sc_v, NEG_BIG)
            m_a2 = jnp.maximum(m_a, sv)
            s_a2 = (s_a * jnp.exp(m_a - m_a2)
                    + jnp.where(valid, jnp.exp(sv - m_a2), 0.0))
            pv = jnp.where(posm, sc_v, NEG_BIG)
            m_p2 = jnp.maximum(m_p, pv)
            s_p2 = (s_p * jnp.exp(m_p - m_p2)
                    + jnp.where(posm, jnp.exp(pv - m_p2), 0.0))
            pc2 = pc + jnp.where(posm, 1.0, 0.0)

            negv = jnp.where(valid & (~posm), vraw, NEG_INF)
            kth = lax.reduce_min(top, (0,))

            def merge(t):
                asc = plsc.sort_key_val(negv, negv)[0]
                bit = jnp.maximum(t, asc)
                return plsc.sort_key_val(bit, bit, descending=True)[0]

            top2 = lax.cond(jnp.any(negv > kth), merge, lambda t: t, top)
            return (m_a2, s_a2, m_p2, s_p2, pc2, top2)

        init = (jnp.full((16,), NEG_BIG, jnp.float32),
                jnp.zeros((16,), jnp.float32),
                jnp.full((16,), NEG_BIG, jnp.float32),
                jnp.zeros((16,), jnp.float32),
                jnp.zeros((16,), jnp.float32),
                jnp.full((16,), NEG_INF, jnp.float32))
        m_a, s_a, m_p, s_p, pc, top = lax.fori_loop(i0, i1, elem_body, init)

        slot = g * 16 + lane
        plsc.store_scatter(v_st, [slot + 0 * G * K], m_a)
        plsc.store_scatter(v_st, [slot + 1 * G * K], s_a)
        plsc.store_scatter(v_st, [slot + 2 * G * K], m_p)
        plsc.store_scatter(v_st, [slot + 3 * G * K], s_p)
        plsc.store_scatter(v_st, [slot + 4 * G * K], pc)
        nf = (end - start).astype(jnp.float32)
        cnt_vec = jnp.where(lane == 0, nf, 0.0)
        plsc.store_scatter(v_st, [slot + 5 * G * K], cnt_vec)
        plsc.store_scatter(v_st, [slot + 6 * G * K], top)
        return 0

    lax.fori_loop(0, G, graph_body, 0)

    pltpu.sync_copy(v_st, o_st.at[pl.ds(wid * WSTAT, WSTAT)])


def _softplus(x):
    return jnp.maximum(x, 0.0) + jnp.log(1.0 + jnp.exp(-jnp.abs(x)))


def _tc_body(lg_ref, tg_ref, eb_ref,
             ma_ref, sa_ref, mp_ref, sp_ref, pc_ref, ct_ref, hnc_ref,
             out_ref,
             hn_ref, vm_ref, pos_ref, nv_ref, accum_ref, listw_ref):
    pid = pl.program_id(0)

    def _red(x, op):
        return op(op(x, axis=2, keepdims=True), axis=0, keepdims=True)

    @pl.when(pid == 0)
    def _init():
        m_ = ma_ref[...]
        s_ = sa_ref[...]
        M3 = _red(m_, jnp.max)
        S3 = _red(s_ * jnp.exp(m_ - M3), jnp.sum)
        mp_ = mp_ref[...]
        sp_ = sp_ref[...]
        Mp3 = _red(mp_, jnp.max)
        Sp3 = _red(sp_ * jnp.exp(mp_ - Mp3), jnp.sum)
        M = M3.reshape(G, 1)
        S = S3.reshape(G, 1)
        Mp = Mp3.reshape(G, 1)
        Sp = Sp3.reshape(G, 1)
        Pos = _red(pc_ref[...], jnp.sum).reshape(G, 1)
        Cnt = _red(ct_ref[...], jnp.sum).reshape(G, 1)
        log_denom = jnp.where(Cnt > 0, M + jnp.log(jnp.maximum(S, TINY)), 0.0)
        log_num = Mp + jnp.log(jnp.maximum(Sp, TINY))
        has_pos = Pos > 0
        log_num_safe = jnp.where(has_pos, log_num, log_denom)
        listwise_sum = jnp.sum(-(log_num_safe - log_denom))
        listwise_den = jnp.maximum(jnp.sum(has_pos.astype(jnp.float32)), 1.0)
        listw_ref[0] = listwise_sum / listwise_den

        cand = hnc_ref[...]
        iw = lax.broadcasted_iota(jnp.int32, (NW, G, K), 0)
        ij = lax.broadcasted_iota(jnp.int32, (NW, G, K), 2)
        flat = iw * K + ij
        cols = []
        for _ in range(K):
            mj = _red(cand, jnp.max)
            first = _red(jnp.where(cand == mj, flat, NW * K), jnp.min)
            cand = jnp.where(flat == first, NEG_INF, cand)
            cols.append(mj.reshape(G, 1))
        hn = jnp.concatenate(cols, axis=1)
        valid = hn > -1e37
        hn_ref[...] = jnp.where(valid, hn, 0.0)
        vm_ref[...] = valid.astype(jnp.float32)
        pos_ref[...] = Pos
        nv_ref[...] = jnp.sum(valid.astype(jnp.float32), axis=1, keepdims=True)
        accum_ref[...] = jnp.zeros((G, 1), jnp.float32)

    v = lg_ref[...]
    posm = tg_ref[...] > 0.5
    eb = eb_ref[...]
    g_lo = jnp.min(eb)
    g_hi = jnp.max(eb)
    giota = lax.broadcasted_iota(jnp.int32, (G, 1), 0)

    def g_body(g, contrib):
        hn_row = hn_ref[pl.ds(g, 1), :]
        vm_row = vm_ref[pl.ds(g, 1), :]
        acc = jnp.zeros_like(v)
        for j in range(K):
            hnj = lax.broadcast_in_dim(hn_row[:, j:j + 1], v.shape, (0, 1))
            vmj = lax.broadcast_in_dim(vm_row[:, j:j + 1], v.shape, (0, 1))
            acc = acc + _softplus(MARGIN + hnj - v) * vmj
        ssum = jnp.sum(jnp.where(posm & (eb == g), acc, 0.0))
        return contrib + jnp.where(giota == g, ssum, 0.0)

    contrib = lax.fori_loop(g_lo, g_hi + 1, g_body,
                            jnp.zeros((G, 1), jnp.float32))
    accum_ref[...] += contrib

    @pl.when(pid == NB - 1)
    def _fin():
        Pos = pos_ref[...]
        nv = nv_ref[...]
        pair_sum = accum_ref[...]
        pair_cnt = Pos * nv
        cond = (Pos > 0) & (nv > 0)
        mean_g = jnp.where(cond, pair_sum / jnp.maximum(pair_cnt, 1.0), 0.0)
        pgraphs = jnp.sum(cond.astype(jnp.float32))
        pairwise = jnp.sum(mean_g) / jnp.maximum(pgraphs, 1.0)
        out_ref[...] = jnp.full((1, 1), listw_ref[0] + PAIR_W * pairwise,
                                jnp.float32)


_tc_part = pl.pallas_call(
    _tc_body,
    grid=(NB,),
    in_specs=(
        [pl.BlockSpec((BR, 128), lambda i: (i, 0))] * 3
        + [pl.BlockSpec((NW, G, K), lambda i: (0, 0, 0))] * 7
    ),
    out_specs=pl.BlockSpec((1, 1), lambda i: (0, 0)),
    out_shape=jax.ShapeDtypeStruct((1, 1), jnp.float32),
    scratch_shapes=[
        pltpu.VMEM((G, K), jnp.float32),
        pltpu.VMEM((G, K), jnp.float32),
        pltpu.VMEM((G, 1), jnp.float32),
        pltpu.VMEM((G, 1), jnp.float32),
        pltpu.VMEM((G, 1), jnp.float32),
        pltpu.SMEM((1,), jnp.float32),
    ],
)


def kernel(logits, targets, edge_batch, num_graphs):
    eb = jnp.minimum(edge_batch.astype(jnp.int32), G - 1)
    st = _sc_part(logits, targets, eb).reshape(NW, NSTAT, G, K)
    ma, sa, mp, sp, pc, ct, hnc = (st[:, i] for i in range(NSTAT))

    padn = E_PAD - E
    lg2 = jnp.concatenate([logits, jnp.zeros((padn,), jnp.float32)])
    tg2 = jnp.concatenate([targets, jnp.zeros((padn,), jnp.float32)])
    eb2 = jnp.concatenate([eb, jnp.full((padn,), G - 1, jnp.int32)])
    out = _tc_part(lg2.reshape(ROWS, 128), tg2.reshape(ROWS, 128),
                   eb2.reshape(ROWS, 128), ma, sa, mp, sp, pc, ct, hnc)
    return out.reshape(())


# speedup vs baseline: 34.6047x; 1.1174x over previous
import functools

import jax
import jax.numpy as jnp
from jax import lax
from jax.experimental import pallas as pl
from jax.experimental.pallas import tpu as pltpu
from jax.experimental.pallas import tpu_sc as plsc

E = 800000
G = 64
NW = 32
CH = E // NW
CHP = CH + 8
K = 16
INV_TEMP = 20.0
MARGIN = 0.2
PAIR_W = 0.3
NEG_BIG = -1e30
NEG_INF = float("-inf")
TINY = 1.1754943508222875e-38

BR = 16
E_PAD = 800768
ROWS = E_PAD // 128
NB = ROWS // BR


_mesh = plsc.VectorSubcoreMesh(core_axis_name="c", subcore_axis_name="s")

NSTAT = 7
WSTAT = NSTAT * G * K


@functools.partial(
    pl.kernel,
    out_type=jax.ShapeDtypeStruct((NW * WSTAT,), jnp.float32),
    mesh=_mesh,
    compiler_params=pltpu.CompilerParams(needs_layout_passes=False),
    scratch_types=[
        pltpu.VMEM((CHP,), jnp.float32),
        pltpu.VMEM((CHP,), jnp.float32),
        pltpu.VMEM((CHP,), jnp.int32),
        pltpu.VMEM((80,), jnp.int32),
        pltpu.VMEM((WSTAT,), jnp.float32),
    ],
)
def _sc_part(logits_hbm, targets_hbm, eb_hbm, o_st, lg_v, tg_v, eb_v, lb_v, v_st):
    c = lax.axis_index("c")
    s = lax.axis_index("s")
    wid = s * 2 + c
    base = wid * CH
    pltpu.sync_copy(logits_hbm.at[pl.ds(base, CH)], lg_v.at[pl.ds(0, CH)])
    pltpu.sync_copy(targets_hbm.at[pl.ds(base, CH)], tg_v.at[pl.ds(0, CH)])
    pltpu.sync_copy(eb_hbm.at[pl.ds(base, CH)], eb_v.at[pl.ds(0, CH)])

    lane = lax.iota(jnp.int32, 16)

    for r in range(4):
        gvec = lane + r * 16

        def bs_body(i, carry):
            lo, hi = carry
            active = lo < hi
            mid = lax.div(lo + hi, 2)
            vals = plsc.load_gather(eb_v, [mid])
            right = vals < gvec
            lo2 = jnp.where(active & right, mid + 1, lo)
            hi2 = jnp.where(active & (~right), mid, hi)
            return lo2, hi2

        lo, hi = lax.fori_loop(
            0, 15, bs_body,
            (jnp.zeros((16,), jnp.int32), jnp.full((16,), CH, jnp.int32)))
        lb_v[pl.ds(r * 16, 16)] = lo
    lb_v[pl.ds(64, 16)] = jnp.full((16,), CH, jnp.int32)

    def graph_body(g, _):
        gfull = jnp.full((16,), g, jnp.int32)
        start_v = plsc.load_gather(lb_v, [gfull])
        end_v = plsc.load_gather(lb_v, [gfull + 1])
        start = lax.reduce_max(start_v, (0,))
        end = lax.reduce_max(end_v, (0,))
        i0 = lax.div(start, 16)
        i1 = lax.div(end + 15, 16)

        def elem_body(i, carry):
            m_a, s_a, m_p, s_p, pc, top = carry
            off = i * 16
            vraw = lg_v[pl.ds(off, 16)]
            trg = tg_v[pl.ds(off, 16)]
            gidx = off + lane
            valid = (gidx >= start) & (gidx < end)
            posm = valid & (trg > 0.5)
            sc_v = vraw * INV_TEMP
            sv = jnp.where(valid, sc_v, NEG_BIG)
            m_a2 = jnp.maximum(m_a, sv)
            s_a2 = (s_a * jnp.exp(m_a - m_a2)
                    + jnp.where(valid, jnp.exp(sv - m_a2), 0.0))
            pv = jnp.where(posm, sc_v, NEG_BIG)
            m_p2 = jnp.maximum(m_p, pv)
            s_p2 = (s_p * jnp.exp(m_p - m_p2)
                    + jnp.where(posm, jnp.exp(pv - m_p2), 0.0))
            pc2 = pc + jnp.where(posm, 1.0, 0.0)

            negv = jnp.where(valid & (~posm), vraw, NEG_INF)
            kth = lax.reduce_min(top, (0,))

            def merge(t):
                asc = plsc.sort_key_val(negv, negv)[0]
                bit = jnp.maximum(t, asc)
                return plsc.sort_key_val(bit, bit, descending=True)[0]

            top2 = lax.cond(jnp.any(negv > kth), merge, lambda t: t, top)
            return (m_a2, s_a2, m_p2, s_p2, pc2, top2)

        init = (jnp.full((16,), NEG_BIG, jnp.float32),
                jnp.zeros((16,), jnp.float32),
                jnp.full((16,), NEG_BIG, jnp.float32),
                jnp.zeros((16,), jnp.float32),
                jnp.zeros((16,), jnp.float32),
                jnp.full((16,), NEG_INF, jnp.float32))
        m_a, s_a, m_p, s_p, pc, top = lax.fori_loop(i0, i1, elem_body, init)

        slot = g * 16 + lane
        plsc.store_scatter(v_st, [slot + 0 * G * K], m_a)
        plsc.store_scatter(v_st, [slot + 1 * G * K], s_a)
        plsc.store_scatter(v_st, [slot + 2 * G * K], m_p)
        plsc.store_scatter(v_st, [slot + 3 * G * K], s_p)
        plsc.store_scatter(v_st, [slot + 4 * G * K], pc)
        nf = (end - start).astype(jnp.float32)
        cnt_vec = jnp.where(lane == 0, nf, 0.0)
        plsc.store_scatter(v_st, [slot + 5 * G * K], cnt_vec)
        plsc.store_scatter(v_st, [slot + 6 * G * K], top)
        return 0

    lax.fori_loop(0, G, graph_body, 0)

    pltpu.sync_copy(v_st, o_st.at[pl.ds(wid * WSTAT, WSTAT)])


def _softplus(x):
    return jnp.maximum(x, 0.0) + jnp.log(1.0 + jnp.exp(-jnp.abs(x)))


def _tc_body(lg_ref, tg_ref, eb_ref,
             ma_ref, sa_ref, mp_ref, sp_ref, pc_ref, ct_ref, hnc_ref,
             out_ref,
             rep_ref, pos_ref, nv_ref, accum_ref, listw_ref):
    pid = pl.program_id(0)

    @pl.when(pid == 0)
    def _init():
        m_ = ma_ref[...]
        s_ = sa_ref[...]
        M = jnp.max(m_, axis=1, keepdims=True)
        S = jnp.sum(s_ * jnp.exp(m_ - M), axis=1, keepdims=True)
        mp_ = mp_ref[...]
        sp_ = sp_ref[...]
        Mp = jnp.max(mp_, axis=1, keepdims=True)
        Sp = jnp.sum(sp_ * jnp.exp(mp_ - Mp), axis=1, keepdims=True)
        Pos = jnp.sum(pc_ref[...], axis=1, keepdims=True)
        Cnt = jnp.sum(ct_ref[...], axis=1, keepdims=True)
        log_denom = jnp.where(Cnt > 0, M + jnp.log(jnp.maximum(S, TINY)), 0.0)
        log_num = Mp + jnp.log(jnp.maximum(Sp, TINY))
        has_pos = Pos > 0
        log_num_safe = jnp.where(has_pos, log_num, log_denom)
        listwise_sum = jnp.sum(-(log_num_safe - log_denom))
        listwise_den = jnp.maximum(jnp.sum(has_pos.astype(jnp.float32)), 1.0)
        listw_ref[0] = listwise_sum / listwise_den

        cand = hnc_ref[...]
        iota1 = lax.broadcasted_iota(jnp.int32, (G, NW * K), 1)
        nv = jnp.zeros((G, 1), jnp.float32)
        for j in range(K):
            mj = jnp.max(cand, axis=1, keepdims=True)
            first = jnp.min(jnp.where(cand == mj, iota1, NW * K),
                            axis=1, keepdims=True)
            cand = jnp.where(iota1 == first, NEG_INF, cand)
            vj = mj > -1e37
            nv += vj.astype(jnp.float32)
            eff = jnp.where(vj, mj + MARGIN, -1e30)
            rep_ref[:, j, :] = lax.broadcast_in_dim(eff, (G, 128), (0, 1))
        pos_ref[...] = Pos
        nv_ref[...] = nv
        accum_ref[...] = jnp.zeros((G, 1), jnp.float32)

    v = lg_ref[...]
    posm = tg_ref[...] > 0.5
    eb = eb_ref[...]
    g_lo = jnp.min(eb)
    g_hi = jnp.max(eb)
    giota = lax.broadcasted_iota(jnp.int32, (G, 1), 0)

    def g_body(g, contrib):
        rows = rep_ref[g]
        acc = jnp.zeros_like(v)
        for j in range(K):
            hb = lax.broadcast_in_dim(rows[j:j + 1, :], v.shape, (0, 1))
            acc = acc + _softplus(hb - v)
        ssum = jnp.sum(jnp.where(posm & (eb == g), acc, 0.0))
        return contrib + jnp.where(giota == g, ssum, 0.0)

    contrib = lax.fori_loop(g_lo, g_hi + 1, g_body,
                            jnp.zeros((G, 1), jnp.float32))
    accum_ref[...] += contrib

    @pl.when(pid == NB - 1)
    def _fin():
        Pos = pos_ref[...]
        nv = nv_ref[...]
        pair_sum = accum_ref[...]
        pair_cnt = Pos * nv
        cond = (Pos > 0) & (nv > 0)
        mean_g = jnp.where(cond, pair_sum / jnp.maximum(pair_cnt, 1.0), 0.0)
        pgraphs = jnp.sum(cond.astype(jnp.float32))
        pairwise = jnp.sum(mean_g) / jnp.maximum(pgraphs, 1.0)
        out_ref[...] = jnp.full((1, 1), listw_ref[0] + PAIR_W * pairwise,
                                jnp.float32)


_tc_part = pl.pallas_call(
    _tc_body,
    grid=(NB,),
    in_specs=(
        [pl.BlockSpec((BR, 128), lambda i: (i, 0))] * 3
        + [pl.BlockSpec((G, NW * K), lambda i: (0, 0))] * 7
    ),
    out_specs=pl.BlockSpec((1, 1), lambda i: (0, 0)),
    out_shape=jax.ShapeDtypeStruct((1, 1), jnp.float32),
    scratch_shapes=[
        pltpu.VMEM((G, K, 128), jnp.float32),
        pltpu.VMEM((G, 1), jnp.float32),
        pltpu.VMEM((G, 1), jnp.float32),
        pltpu.VMEM((G, 1), jnp.float32),
        pltpu.SMEM((1,), jnp.float32),
    ],
)


def kernel(logits, targets, edge_batch, num_graphs):
    eb = jnp.minimum(edge_batch.astype(jnp.int32), G - 1)
    st = _sc_part(logits, targets, eb).reshape(NW, NSTAT, G, K)
    st = jnp.transpose(st, (1, 2, 0, 3)).reshape(NSTAT, G, NW * K)
    ma, sa, mp, sp, pc, ct, hnc = (st[i] for i in range(NSTAT))

    padn = E_PAD - E
    lg2 = jnp.concatenate([logits, jnp.zeros((padn,), jnp.float32)])
    tg2 = jnp.concatenate([targets, jnp.zeros((padn,), jnp.float32)])
    eb2 = jnp.concatenate([eb, jnp.full((padn,), G - 1, jnp.int32)])
    out = _tc_part(lg2.reshape(ROWS, 128), tg2.reshape(ROWS, 128),
                   eb2.reshape(ROWS, 128), ma, sa, mp, sp, pc, ct, hnc)
    return out.reshape(())


# speedup vs baseline: 34.7171x; 1.0032x over previous
import functools

import jax
import jax.numpy as jnp
from jax import lax
from jax.experimental import pallas as pl
from jax.experimental.pallas import tpu as pltpu
from jax.experimental.pallas import tpu_sc as plsc

E = 800000
G = 64
NW = 32
CH = E // NW
CHP = CH + 8
K = 16
INV_TEMP = 20.0
MARGIN = 0.2
PAIR_W = 0.3
NEG_BIG = -1e30
NEG_INF = float("-inf")
TINY = 1.1754943508222875e-38

BR = 16
E_PAD = 800768
ROWS = E_PAD // 128
NB = ROWS // BR


_mesh = plsc.VectorSubcoreMesh(core_axis_name="c", subcore_axis_name="s")

NSTAT = 7
WSTAT = NSTAT * G * K


@functools.partial(
    pl.kernel,
    out_type=jax.ShapeDtypeStruct((NW * WSTAT,), jnp.float32),
    mesh=_mesh,
    compiler_params=pltpu.CompilerParams(needs_layout_passes=False),
    scratch_types=[
        pltpu.VMEM((CHP,), jnp.float32),
        pltpu.VMEM((CHP,), jnp.float32),
        pltpu.VMEM((CHP,), jnp.int32),
        pltpu.VMEM((80,), jnp.int32),
        pltpu.VMEM((WSTAT,), jnp.float32),
    ],
)
def _sc_part(logits_hbm, targets_hbm, eb_hbm, o_st, lg_v, tg_v, eb_v, lb_v, v_st):
    c = lax.axis_index("c")
    s = lax.axis_index("s")
    wid = s * 2 + c
    base = wid * CH
    pltpu.sync_copy(logits_hbm.at[pl.ds(base, CH)], lg_v.at[pl.ds(0, CH)])
    pltpu.sync_copy(targets_hbm.at[pl.ds(base, CH)], tg_v.at[pl.ds(0, CH)])
    pltpu.sync_copy(eb_hbm.at[pl.ds(base, CH)], eb_v.at[pl.ds(0, CH)])

    lane = lax.iota(jnp.int32, 16)

    for r in range(4):
        gvec = lane + r * 16

        def bs_body(i, carry):
            lo, hi = carry
            active = lo < hi
            mid = lax.div(lo + hi, 2)
            vals = plsc.load_gather(eb_v, [mid])
            right = vals < gvec
            lo2 = jnp.where(active & right, mid + 1, lo)
            hi2 = jnp.where(active & (~right), mid, hi)
            return lo2, hi2

        lo, hi = lax.fori_loop(
            0, 15, bs_body,
            (jnp.zeros((16,), jnp.int32), jnp.full((16,), CH, jnp.int32)))
        lb_v[pl.ds(r * 16, 16)] = lo
    lb_v[pl.ds(64, 16)] = jnp.full((16,), CH, jnp.int32)

    def graph_body(g, _):
        gfull = jnp.full((16,), g, jnp.int32)
        start_v = plsc.load_gather(lb_v, [gfull])
        end_v = plsc.load_gather(lb_v, [gfull + 1])
        start = lax.reduce_max(start_v, (0,))
        end = lax.reduce_max(end_v, (0,))
        i0 = lax.div(start, 16)
        i1 = lax.div(end + 15, 16)

        def elem_body(i, carry):
            m_a, s_a, m_p, s_p, pc, top = carry
            off = i * 16
            vraw = lg_v[pl.ds(off, 16)]
            trg = tg_v[pl.ds(off, 16)]
            gidx = off + lane
            valid = (gidx >= start) & (gidx < end)
            posm = valid & (trg > 0.5)
            sc_v = vraw * INV_TEMP
            sv = jnp.where(valid, sc_v, NEG_BIG)
            m_a2 = jnp.maximum(m_a, sv)
            s_a2 = (s_a * jnp.exp(m_a - m_a2)
                    + jnp.where(valid, jnp.exp(sv - m_a2), 0.0))
            pv = jnp.where(posm, sc_v, NEG_BIG)
            m_p2 = jnp.maximum(m_p, pv)
            s_p2 = (s_p * jnp.exp(m_p - m_p2)
                    + jnp.where(posm, jnp.exp(pv - m_p2), 0.0))
            pc2 = pc + jnp.where(posm, 1.0, 0.0)

            negv = jnp.where(valid & (~posm), vraw, NEG_INF)
            kth = lax.reduce_min(top, (0,))

            def merge(t):
                asc = plsc.sort_key_val(negv, negv)[0]
                bit = jnp.maximum(t, asc)
                return plsc.sort_key_val(bit, bit, descending=True)[0]

            top2 = lax.cond(jnp.any(negv > kth), merge, lambda t: t, top)
            return (m_a2, s_a2, m_p2, s_p2, pc2, top2)

        init = (jnp.full((16,), NEG_BIG, jnp.float32),
                jnp.zeros((16,), jnp.float32),
                jnp.full((16,), NEG_BIG, jnp.float32),
                jnp.zeros((16,), jnp.float32),
                jnp.zeros((16,), jnp.float32),
                jnp.full((16,), NEG_INF, jnp.float32))
        m_a, s_a, m_p, s_p, pc, top = lax.fori_loop(i0, i1, elem_body, init)

        slot = g * 16 + lane
        plsc.store_scatter(v_st, [slot + 0 * G * K], m_a)
        plsc.store_scatter(v_st, [slot + 1 * G * K], s_a)
        plsc.store_scatter(v_st, [slot + 2 * G * K], m_p)
        plsc.store_scatter(v_st, [slot + 3 * G * K], s_p)
        plsc.store_scatter(v_st, [slot + 4 * G * K], pc)
        nf = (end - start).astype(jnp.float32)
        cnt_vec = jnp.where(lane == 0, nf, 0.0)
        plsc.store_scatter(v_st, [slot + 5 * G * K], cnt_vec)
        plsc.store_scatter(v_st, [slot + 6 * G * K], top)
        return 0

    lax.fori_loop(0, G, graph_body, 0)

    pltpu.sync_copy(v_st, o_st.at[pl.ds(wid * WSTAT, WSTAT)])


def _softplus(x):
    return jnp.maximum(x, 0.0) + jnp.log(1.0 + jnp.exp(-jnp.abs(x)))


def _tc_body(lg_ref, tg_ref, eb_ref,
             ma_ref, sa_ref, mp_ref, sp_ref, pc_ref, ct_ref, hnc_ref,
             out_ref,
             rep_ref, pos_ref, nv_ref, accum_ref, listw_ref):
    pid = pl.program_id(0)

    @pl.when(pid == 0)
    def _init():
        m_ = ma_ref[...]
        s_ = sa_ref[...]
        M = jnp.max(m_, axis=1, keepdims=True)
        S = jnp.sum(s_ * jnp.exp(m_ - M), axis=1, keepdims=True)
        mp_ = mp_ref[...]
        sp_ = sp_ref[...]
        Mp = jnp.max(mp_, axis=1, keepdims=True)
        Sp = jnp.sum(sp_ * jnp.exp(mp_ - Mp), axis=1, keepdims=True)
        Pos = jnp.sum(pc_ref[...], axis=1, keepdims=True)
        Cnt = jnp.sum(ct_ref[...], axis=1, keepdims=True)
        log_denom = jnp.where(Cnt > 0, M + jnp.log(jnp.maximum(S, TINY)), 0.0)
        log_num = Mp + jnp.log(jnp.maximum(Sp, TINY))
        has_pos = Pos > 0
        log_num_safe = jnp.where(has_pos, log_num, log_denom)
        listwise_sum = jnp.sum(-(log_num_safe - log_denom))
        listwise_den = jnp.maximum(jnp.sum(has_pos.astype(jnp.float32)), 1.0)
        listw_ref[0] = listwise_sum / listwise_den

        cand = hnc_ref[...]
        iota1 = lax.broadcasted_iota(jnp.int32, (G, NW * K), 1)
        nv = jnp.zeros((G, 1), jnp.float32)
        for j in range(K):
            mj = jnp.max(cand, axis=1, keepdims=True)
            first = jnp.min(jnp.where(cand == mj, iota1, NW * K),
                            axis=1, keepdims=True)
            cand = jnp.where(iota1 == first, NEG_INF, cand)
            vj = mj > -1e37
            nv += vj.astype(jnp.float32)
            eff = jnp.where(vj, mj + MARGIN, -1e30)
            rep_ref[:, j] = lax.broadcast_in_dim(eff, (G, BR, 128), (0, 1))
        pos_ref[...] = Pos
        nv_ref[...] = nv
        accum_ref[...] = jnp.zeros((G, 1), jnp.float32)

    v = lg_ref[...]
    posm = tg_ref[...] > 0.5
    eb = eb_ref[...]
    g_lo = jnp.min(eb)
    g_hi = jnp.max(eb)
    giota = lax.broadcasted_iota(jnp.int32, (G, 1), 0)

    def g_body(g, contrib):
        x3 = rep_ref[g]
        v3 = lax.broadcast_in_dim(v, (K, BR, 128), (1, 2))
        acc = jnp.sum(_softplus(x3 - v3), axis=0)
        ssum = jnp.sum(jnp.where(posm & (eb == g), acc, 0.0))
        return contrib + jnp.where(giota == g, ssum, 0.0)

    contrib = lax.fori_loop(g_lo, g_hi + 1, g_body,
                            jnp.zeros((G, 1), jnp.float32))
    accum_ref[...] += contrib

    @pl.when(pid == NB - 1)
    def _fin():
        Pos = pos_ref[...]
        nv = nv_ref[...]
        pair_sum = accum_ref[...]
        pair_cnt = Pos * nv
        cond = (Pos > 0) & (nv > 0)
        mean_g = jnp.where(cond, pair_sum / jnp.maximum(pair_cnt, 1.0), 0.0)
        pgraphs = jnp.sum(cond.astype(jnp.float32))
        pairwise = jnp.sum(mean_g) / jnp.maximum(pgraphs, 1.0)
        out_ref[...] = jnp.full((1, 1), listw_ref[0] + PAIR_W * pairwise,
                                jnp.float32)


_tc_part = pl.pallas_call(
    _tc_body,
    grid=(NB,),
    in_specs=(
        [pl.BlockSpec((BR, 128), lambda i: (i, 0))] * 3
        + [pl.BlockSpec((G, NW * K), lambda i: (0, 0))] * 7
    ),
    out_specs=pl.BlockSpec((1, 1), lambda i: (0, 0)),
    out_shape=jax.ShapeDtypeStruct((1, 1), jnp.float32),
    scratch_shapes=[
        pltpu.VMEM((G, K, BR, 128), jnp.float32),
        pltpu.VMEM((G, 1), jnp.float32),
        pltpu.VMEM((G, 1), jnp.float32),
        pltpu.VMEM((G, 1), jnp.float32),
        pltpu.SMEM((1,), jnp.float32),
    ],
)


def kernel(logits, targets, edge_batch, num_graphs):
    eb = jnp.minimum(edge_batch.astype(jnp.int32), G - 1)
    st = _sc_part(logits, targets, eb).reshape(NW, NSTAT, G, K)
    st = jnp.transpose(st, (1, 2, 0, 3)).reshape(NSTAT, G, NW * K)
    ma, sa, mp, sp, pc, ct, hnc = (st[i] for i in range(NSTAT))

    padn = E_PAD - E
    lg2 = jnp.concatenate([logits, jnp.zeros((padn,), jnp.float32)])
    tg2 = jnp.concatenate([targets, jnp.zeros((padn,), jnp.float32)])
    eb2 = jnp.concatenate([eb, jnp.full((padn,), G - 1, jnp.int32)])
    out = _tc_part(lg2.reshape(ROWS, 128), tg2.reshape(ROWS, 128),
                   eb2.reshape(ROWS, 128), ma, sa, mp, sp, pc, ct, hnc)
    return out.reshape(())


# speedup vs baseline: 65.7629x; 1.8943x over previous
import functools

import jax
import jax.numpy as jnp
from jax import lax
from jax.experimental import pallas as pl
from jax.experimental.pallas import tpu as pltpu
from jax.experimental.pallas import tpu_sc as plsc

E = 800000
G = 64
NW = 32
CH = E // NW
CHP = CH + 8
K = 16
INV_TEMP = 20.0
MARGIN = 0.2
PAIR_W = 0.3
NEG_BIG = -1e30
NEG_INF = float("-inf")
TINY = 1.1754943508222875e-38

BR = 64
SUB = 16
E_PAD = 802816
ROWS = E_PAD // 128
NB = ROWS // BR


_mesh = plsc.VectorSubcoreMesh(core_axis_name="c", subcore_axis_name="s")

NSTAT = 7
WSTAT = NSTAT * G * K


@functools.partial(
    pl.kernel,
    out_type=jax.ShapeDtypeStruct((NW * WSTAT,), jnp.float32),
    mesh=_mesh,
    compiler_params=pltpu.CompilerParams(needs_layout_passes=False),
    scratch_types=[
        pltpu.VMEM((CHP,), jnp.float32),
        pltpu.VMEM((CHP,), jnp.float32),
        pltpu.VMEM((CHP,), jnp.int32),
        pltpu.VMEM((80,), jnp.int32),
        pltpu.VMEM((WSTAT,), jnp.float32),
    ],
)
def _sc_part(logits_hbm, targets_hbm, eb_hbm, o_st, lg_v, tg_v, eb_v, lb_v, v_st):
    c = lax.axis_index("c")
    s = lax.axis_index("s")
    wid = s * 2 + c
    base = wid * CH
    pltpu.sync_copy(logits_hbm.at[pl.ds(base, CH)], lg_v.at[pl.ds(0, CH)])
    pltpu.sync_copy(targets_hbm.at[pl.ds(base, CH)], tg_v.at[pl.ds(0, CH)])
    pltpu.sync_copy(eb_hbm.at[pl.ds(base, CH)], eb_v.at[pl.ds(0, CH)])

    lane = lax.iota(jnp.int32, 16)

    for r in range(4):
        gvec = lane + r * 16

        def bs_body(i, carry):
            lo, hi = carry
            active = lo < hi
            mid = lax.div(lo + hi, 2)
            vals = plsc.load_gather(eb_v, [mid])
            right = vals < gvec
            lo2 = jnp.where(active & right, mid + 1, lo)
            hi2 = jnp.where(active & (~right), mid, hi)
            return lo2, hi2

        lo, hi = lax.fori_loop(
            0, 15, bs_body,
            (jnp.zeros((16,), jnp.int32), jnp.full((16,), CH, jnp.int32)))
        lb_v[pl.ds(r * 16, 16)] = lo
    lb_v[pl.ds(64, 16)] = jnp.full((16,), CH, jnp.int32)

    def graph_body(g, _):
        gfull = jnp.full((16,), g, jnp.int32)
        start_v = plsc.load_gather(lb_v, [gfull])
        end_v = plsc.load_gather(lb_v, [gfull + 1])
        start = lax.reduce_max(start_v, (0,))
        end = lax.reduce_max(end_v, (0,))
        i0 = lax.div(start, 16)
        i1 = lax.div(end + 15, 16)

        def elem_body(i, carry):
            m_a, s_a, m_p, s_p, pc, top = carry
            off = i * 16
            vraw = lg_v[pl.ds(off, 16)]
            trg = tg_v[pl.ds(off, 16)]
            gidx = off + lane
            valid = (gidx >= start) & (gidx < end)
            posm = valid & (trg > 0.5)
            sc_v = vraw * INV_TEMP
            sv = jnp.where(valid, sc_v, NEG_BIG)
            m_a2 = jnp.maximum(m_a, sv)
            s_a2 = (s_a * jnp.exp(m_a - m_a2)
                    + jnp.where(valid, jnp.exp(sv - m_a2), 0.0))
            pv = jnp.where(posm, sc_v, NEG_BIG)
            m_p2 = jnp.maximum(m_p, pv)
            s_p2 = (s_p * jnp.exp(m_p - m_p2)
                    + jnp.where(posm, jnp.exp(pv - m_p2), 0.0))
            pc2 = pc + jnp.where(posm, 1.0, 0.0)

            negv = jnp.where(valid & (~posm), vraw, NEG_INF)
            kth = lax.reduce_min(top, (0,))

            def merge(t):
                asc = plsc.sort_key_val(negv, negv)[0]
                bit = jnp.maximum(t, asc)
                return plsc.sort_key_val(bit, bit, descending=True)[0]

            top2 = lax.cond(jnp.any(negv > kth), merge, lambda t: t, top)
            return (m_a2, s_a2, m_p2, s_p2, pc2, top2)

        init = (jnp.full((16,), NEG_BIG, jnp.float32),
                jnp.zeros((16,), jnp.float32),
                jnp.full((16,), NEG_BIG, jnp.float32),
                jnp.zeros((16,), jnp.float32),
                jnp.zeros((16,), jnp.float32),
                jnp.full((16,), NEG_INF, jnp.float32))
        m_a, s_a, m_p, s_p, pc, top = lax.fori_loop(i0, i1, elem_body, init)

        slot = g * 16 + lane
        plsc.store_scatter(v_st, [slot + 0 * G * K], m_a)
        plsc.store_scatter(v_st, [slot + 1 * G * K], s_a)
        plsc.store_scatter(v_st, [slot + 2 * G * K], m_p)
        plsc.store_scatter(v_st, [slot + 3 * G * K], s_p)
        plsc.store_scatter(v_st, [slot + 4 * G * K], pc)
        nf = (end - start).astype(jnp.float32)
        cnt_vec = jnp.where(lane == 0, nf, 0.0)
        plsc.store_scatter(v_st, [slot + 5 * G * K], cnt_vec)
        plsc.store_scatter(v_st, [slot + 6 * G * K], top)
        return 0

    lax.fori_loop(0, G, graph_body, 0)

    pltpu.sync_copy(v_st, o_st.at[pl.ds(wid * WSTAT, WSTAT)])


def _softplus(x):
    return jnp.maximum(x, 0.0) + jnp.log(1.0 + jnp.exp(-jnp.abs(x)))


def _tc_body(lg_ref, tg_ref, eb_ref,
             ma_ref, sa_ref, mp_ref, sp_ref, pc_ref, ct_ref, hnc_ref,
             out_ref,
             rep_ref, pos_ref, nv_ref, accum_ref, listw_ref):
    pid = pl.program_id(0)

    @pl.when(pid == 0)
    def _init():
        m_ = ma_ref[...]
        s_ = sa_ref[...]
        M = jnp.max(m_, axis=1, keepdims=True)
        S = jnp.sum(s_ * jnp.exp(m_ - M), axis=1, keepdims=True)
        mp_ = mp_ref[...]
        sp_ = sp_ref[...]
        Mp = jnp.max(mp_, axis=1, keepdims=True)
        Sp = jnp.sum(sp_ * jnp.exp(mp_ - Mp), axis=1, keepdims=True)
        Pos = jnp.sum(pc_ref[...], axis=1, keepdims=True)
        Cnt = jnp.sum(ct_ref[...], axis=1, keepdims=True)
        log_denom = jnp.where(Cnt > 0, M + jnp.log(jnp.maximum(S, TINY)), 0.0)
        log_num = Mp + jnp.log(jnp.maximum(Sp, TINY))
        has_pos = Pos > 0
        log_num_safe = jnp.where(has_pos, log_num, log_denom)
        listwise_sum = jnp.sum(-(log_num_safe - log_denom))
        listwise_den = jnp.maximum(jnp.sum(has_pos.astype(jnp.float32)), 1.0)
        listw_ref[0] = listwise_sum / listwise_den

        cand = hnc_ref[...]
        iota1 = lax.broadcasted_iota(jnp.int32, (G, NW * K), 1)
        nv = jnp.zeros((G, 1), jnp.float32)
        for j in range(K):
            mj = jnp.max(cand, axis=1, keepdims=True)
            first = jnp.min(jnp.where(cand == mj, iota1, NW * K),
                            axis=1, keepdims=True)
            cand = jnp.where(iota1 == first, NEG_INF, cand)
            vj = mj > -1e37
            nv += vj.astype(jnp.float32)
            eff = jnp.where(vj, mj + MARGIN, -1e30)
            rep_ref[:, j] = lax.broadcast_in_dim(eff, (G, SUB, 128), (0, 1))
        pos_ref[...] = Pos
        nv_ref[...] = nv
        accum_ref[...] = jnp.zeros((G, 1), jnp.float32)

    v = lg_ref[...]
    posm = tg_ref[...] > 0.5
    eb = eb_ref[...]
    g_lo = jnp.min(eb)
    g_hi = jnp.max(eb)
    giota = lax.broadcasted_iota(jnp.int32, (G, 1), 0)

    def g_body(g, contrib):
        x3 = rep_ref[g]
        msk = posm & (eb == g)
        ssum = jnp.float32(0.0)
        for t in range(BR // SUB):
            vt = v[t * SUB:(t + 1) * SUB, :]
            v3 = lax.broadcast_in_dim(vt, (K, SUB, 128), (1, 2))
            acc = jnp.sum(_softplus(x3 - v3), axis=0)
            ssum += jnp.sum(
                jnp.where(msk[t * SUB:(t + 1) * SUB, :], acc, 0.0))
        return contrib + jnp.where(giota == g, ssum, 0.0)

    contrib = lax.fori_loop(g_lo, g_hi + 1, g_body,
                            jnp.zeros((G, 1), jnp.float32))
    accum_ref[...] += contrib

    @pl.when(pid == NB - 1)
    def _fin():
        Pos = pos_ref[...]
        nv = nv_ref[...]
        pair_sum = accum_ref[...]
        pair_cnt = Pos * nv
        cond = (Pos > 0) & (nv > 0)
        mean_g = jnp.where(cond, pair_sum / jnp.maximum(pair_cnt, 1.0), 0.0)
        pgraphs = jnp.sum(cond.astype(jnp.float32))
        pairwise = jnp.sum(mean_g) / jnp.maximum(pgraphs, 1.0)
        out_ref[...] = jnp.full((1, 1), listw_ref[0] + PAIR_W * pairwise,
                                jnp.float32)


_tc_part = pl.pallas_call(
    _tc_body,
    grid=(NB,),
    in_specs=(
        [pl.BlockSpec((BR, 128), lambda i: (i, 0))] * 3
        + [pl.BlockSpec((G, NW * K), lambda i: (0, 0))] * 7
    ),
    out_specs=pl.BlockSpec((1, 1), lambda i: (0, 0)),
    out_shape=jax.ShapeDtypeStruct((1, 1), jnp.float32),
    scratch_shapes=[
        pltpu.VMEM((G, K, SUB, 128), jnp.float32),
        pltpu.VMEM((G, 1), jnp.float32),
        pltpu.VMEM((G, 1), jnp.float32),
        pltpu.VMEM((G, 1), jnp.float32),
        pltpu.SMEM((1,), jnp.float32),
    ],
)


def kernel(logits, targets, edge_batch, num_graphs):
    eb = jnp.minimum(edge_batch.astype(jnp.int32), G - 1)
    st = _sc_part(logits, targets, eb).reshape(NW, NSTAT, G, K)
    st = jnp.transpose(st, (1, 2, 0, 3)).reshape(NSTAT, G, NW * K)
    ma, sa, mp, sp, pc, ct, hnc = (st[i] for i in range(NSTAT))

    padn = E_PAD - E
    lg2 = jnp.concatenate([logits, jnp.zeros((padn,), jnp.float32)])
    tg2 = jnp.concatenate([targets, jnp.zeros((padn,), jnp.float32)])
    eb2 = jnp.concatenate([eb, jnp.full((padn,), G - 1, jnp.int32)])
    out = _tc_part(lg2.reshape(ROWS, 128), tg2.reshape(ROWS, 128),
                   eb2.reshape(ROWS, 128), ma, sa, mp, sp, pc, ct, hnc)
    return out.reshape(())


# speedup vs baseline: 66.9144x; 1.0175x over previous
import functools

import jax
import jax.numpy as jnp
from jax import lax
from jax.experimental import pallas as pl
from jax.experimental.pallas import tpu as pltpu
from jax.experimental.pallas import tpu_sc as plsc

E = 800000
G = 64
NW = 32
CH = E // NW
CHP = CH + 8
K = 16
INV_TEMP = 20.0
MARGIN = 0.2
PAIR_W = 0.3
NEG_BIG = -1e30
NEG_INF = float("-inf")
TINY = 1.1754943508222875e-38

BR = 64
SUB = 16
ROWS = E // 128
NB = -(-ROWS // BR)


_mesh = plsc.VectorSubcoreMesh(core_axis_name="c", subcore_axis_name="s")

NSTAT = 7
WSTAT = NSTAT * G * K


@functools.partial(
    pl.kernel,
    out_type=jax.ShapeDtypeStruct((NW * WSTAT,), jnp.float32),
    mesh=_mesh,
    compiler_params=pltpu.CompilerParams(needs_layout_passes=False),
    scratch_types=[
        pltpu.VMEM((CHP,), jnp.float32),
        pltpu.VMEM((CHP,), jnp.float32),
        pltpu.VMEM((CHP,), jnp.int32),
        pltpu.VMEM((80,), jnp.int32),
        pltpu.VMEM((WSTAT,), jnp.float32),
    ],
)
def _sc_part(logits_hbm, targets_hbm, eb_hbm, o_st, lg_v, tg_v, eb_v, lb_v, v_st):
    c = lax.axis_index("c")
    s = lax.axis_index("s")
    wid = s * 2 + c
    base = wid * CH
    pltpu.sync_copy(logits_hbm.at[pl.ds(base, CH)], lg_v.at[pl.ds(0, CH)])
    pltpu.sync_copy(targets_hbm.at[pl.ds(base, CH)], tg_v.at[pl.ds(0, CH)])
    pltpu.sync_copy(eb_hbm.at[pl.ds(base, CH)], eb_v.at[pl.ds(0, CH)])

    lane = lax.iota(jnp.int32, 16)

    for r in range(4):
        gvec = lane + r * 16

        def bs_body(i, carry):
            lo, hi = carry
            active = lo < hi
            mid = lax.div(lo + hi, 2)
            vals = plsc.load_gather(eb_v, [mid])
            right = vals < gvec
            lo2 = jnp.where(active & right, mid + 1, lo)
            hi2 = jnp.where(active & (~right), mid, hi)
            return lo2, hi2

        lo, hi = lax.fori_loop(
            0, 15, bs_body,
            (jnp.zeros((16,), jnp.int32), jnp.full((16,), CH, jnp.int32)))
        lb_v[pl.ds(r * 16, 16)] = lo
    lb_v[pl.ds(64, 16)] = jnp.full((16,), CH, jnp.int32)

    def graph_body(g, _):
        gfull = jnp.full((16,), g, jnp.int32)
        start_v = plsc.load_gather(lb_v, [gfull])
        end_v = plsc.load_gather(lb_v, [gfull + 1])
        start = lax.reduce_max(start_v, (0,))
        end = lax.reduce_max(end_v, (0,))
        i0 = lax.div(start, 16)
        i1 = lax.div(end + 15, 16)

        def elem_body(i, carry):
            m_a, s_a, m_p, s_p, pc, top = carry
            off = i * 16
            vraw = lg_v[pl.ds(off, 16)]
            trg = tg_v[pl.ds(off, 16)]
            gidx = off + lane
            valid = (gidx >= start) & (gidx < end)
            posm = valid & (trg > 0.5)
            sc_v = vraw * INV_TEMP
            sv = jnp.where(valid, sc_v, NEG_BIG)
            m_a2 = jnp.maximum(m_a, sv)
            s_a2 = (s_a * jnp.exp(m_a - m_a2)
                    + jnp.where(valid, jnp.exp(sv - m_a2), 0.0))
            pv = jnp.where(posm, sc_v, NEG_BIG)
            m_p2 = jnp.maximum(m_p, pv)
            s_p2 = (s_p * jnp.exp(m_p - m_p2)
                    + jnp.where(posm, jnp.exp(pv - m_p2), 0.0))
            pc2 = pc + jnp.where(posm, 1.0, 0.0)

            negv = jnp.where(valid & (~posm), vraw, NEG_INF)
            kth = lax.reduce_min(top, (0,))

            def merge(t):
                asc = plsc.sort_key_val(negv, negv)[0]
                bit = jnp.maximum(t, asc)
                return plsc.sort_key_val(bit, bit, descending=True)[0]

            top2 = lax.cond(jnp.any(negv > kth), merge, lambda t: t, top)
            return (m_a2, s_a2, m_p2, s_p2, pc2, top2)

        init = (jnp.full((16,), NEG_BIG, jnp.float32),
                jnp.zeros((16,), jnp.float32),
                jnp.full((16,), NEG_BIG, jnp.float32),
                jnp.zeros((16,), jnp.float32),
                jnp.zeros((16,), jnp.float32),
                jnp.full((16,), NEG_INF, jnp.float32))
        m_a, s_a, m_p, s_p, pc, top = lax.fori_loop(i0, i1, elem_body, init)

        slot = g * 16 + lane
        plsc.store_scatter(v_st, [slot + 0 * G * K], m_a)
        plsc.store_scatter(v_st, [slot + 1 * G * K], s_a)
        plsc.store_scatter(v_st, [slot + 2 * G * K], m_p)
        plsc.store_scatter(v_st, [slot + 3 * G * K], s_p)
        plsc.store_scatter(v_st, [slot + 4 * G * K], pc)
        nf = (end - start).astype(jnp.float32)
        cnt_vec = jnp.where(lane == 0, nf, 0.0)
        plsc.store_scatter(v_st, [slot + 5 * G * K], cnt_vec)
        plsc.store_scatter(v_st, [slot + 6 * G * K], top)
        return 0

    lax.fori_loop(0, G, graph_body, 0)

    pltpu.sync_copy(v_st, o_st.at[pl.ds(wid * WSTAT, WSTAT)])


def _softplus(x):
    return jnp.maximum(x, 0.0) + jnp.log(1.0 + jnp.exp(-jnp.abs(x)))


def _tc_body(lg_ref, tg_ref, eb_ref,
             ma_ref, sa_ref, mp_ref, sp_ref, pc_ref, ct_ref, hnc_ref,
             out_ref,
             rep_ref, pos_ref, nv_ref, accum_ref, listw_ref):
    pid = pl.program_id(0)

    @pl.when(pid == 0)
    def _init():
        m_ = ma_ref[...]
        s_ = sa_ref[...]
        M = jnp.max(m_, axis=1, keepdims=True)
        S = jnp.sum(s_ * jnp.exp(m_ - M), axis=1, keepdims=True)
        mp_ = mp_ref[...]
        sp_ = sp_ref[...]
        Mp = jnp.max(mp_, axis=1, keepdims=True)
        Sp = jnp.sum(sp_ * jnp.exp(mp_ - Mp), axis=1, keepdims=True)
        Pos = jnp.sum(pc_ref[...], axis=1, keepdims=True)
        Cnt = jnp.sum(ct_ref[...], axis=1, keepdims=True)
        log_denom = jnp.where(Cnt > 0, M + jnp.log(jnp.maximum(S, TINY)), 0.0)
        log_num = Mp + jnp.log(jnp.maximum(Sp, TINY))
        has_pos = Pos > 0
        log_num_safe = jnp.where(has_pos, log_num, log_denom)
        listwise_sum = jnp.sum(-(log_num_safe - log_denom))
        listwise_den = jnp.maximum(jnp.sum(has_pos.astype(jnp.float32)), 1.0)
        listw_ref[0] = listwise_sum / listwise_den

        cand = hnc_ref[...]
        iota1 = lax.broadcasted_iota(jnp.int32, (G, NW * K), 1)
        nv = jnp.zeros((G, 1), jnp.float32)
        for j in range(K):
            mj = jnp.max(cand, axis=1, keepdims=True)
            first = jnp.min(jnp.where(cand == mj, iota1, NW * K),
                            axis=1, keepdims=True)
            cand = jnp.where(iota1 == first, NEG_INF, cand)
            vj = mj > -1e37
            nv += vj.astype(jnp.float32)
            eff = jnp.where(vj, mj + MARGIN, -1e30)
            rep_ref[:, j] = lax.broadcast_in_dim(eff, (G, SUB, 128), (0, 1))
        pos_ref[...] = Pos
        nv_ref[...] = nv
        accum_ref[...] = jnp.zeros((G, 1), jnp.float32)

    v = lg_ref[...]
    inb = (pid * BR + lax.broadcasted_iota(jnp.int32, (BR, 128), 0)) < ROWS
    posm = (tg_ref[...] > 0.5) & inb
    eb = jnp.where(inb, eb_ref[...], G - 1)
    g_lo = jnp.min(eb)
    g_hi = jnp.max(eb)
    giota = lax.broadcasted_iota(jnp.int32, (G, 1), 0)

    def g_body(g, contrib):
        x3 = rep_ref[g]
        msk = posm & (eb == g)
        ssum = jnp.float32(0.0)
        for t in range(BR // SUB):
            vt = v[t * SUB:(t + 1) * SUB, :]
            v3 = lax.broadcast_in_dim(vt, (K, SUB, 128), (1, 2))
            acc = jnp.sum(_softplus(x3 - v3), axis=0)
            ssum += jnp.sum(
                jnp.where(msk[t * SUB:(t + 1) * SUB, :], acc, 0.0))
        return contrib + jnp.where(giota == g, ssum, 0.0)

    contrib = lax.fori_loop(g_lo, g_hi + 1, g_body,
                            jnp.zeros((G, 1), jnp.float32))
    accum_ref[...] += contrib

    @pl.when(pid == NB - 1)
    def _fin():
        Pos = pos_ref[...]
        nv = nv_ref[...]
        pair_sum = accum_ref[...]
        pair_cnt = Pos * nv
        cond = (Pos > 0) & (nv > 0)
        mean_g = jnp.where(cond, pair_sum / jnp.maximum(pair_cnt, 1.0), 0.0)
        pgraphs = jnp.sum(cond.astype(jnp.float32))
        pairwise = jnp.sum(mean_g) / jnp.maximum(pgraphs, 1.0)
        out_ref[...] = jnp.full((1, 1), listw_ref[0] + PAIR_W * pairwise,
                                jnp.float32)


_tc_part = pl.pallas_call(
    _tc_body,
    grid=(NB,),
    in_specs=(
        [pl.BlockSpec((BR, 128), lambda i: (i, 0))] * 3
        + [pl.BlockSpec((G, NW * K), lambda i: (0, 0))] * 7
    ),
    out_specs=pl.BlockSpec((1, 1), lambda i: (0, 0)),
    out_shape=jax.ShapeDtypeStruct((1, 1), jnp.float32),
    scratch_shapes=[
        pltpu.VMEM((G, K, SUB, 128), jnp.float32),
        pltpu.VMEM((G, 1), jnp.float32),
        pltpu.VMEM((G, 1), jnp.float32),
        pltpu.VMEM((G, 1), jnp.float32),
        pltpu.SMEM((1,), jnp.float32),
    ],
)


def kernel(logits, targets, edge_batch, num_graphs):
    eb = edge_batch.astype(jnp.int32)
    st = _sc_part(logits, targets, eb).reshape(NW, NSTAT, G, K)
    st = jnp.transpose(st, (1, 2, 0, 3)).reshape(NSTAT, G, NW * K)
    ma, sa, mp, sp, pc, ct, hnc = (st[i] for i in range(NSTAT))

    out = _tc_part(logits.reshape(ROWS, 128), targets.reshape(ROWS, 128),
                   eb.reshape(ROWS, 128), ma, sa, mp, sp, pc, ct, hnc)
    return out.reshape(())


# speedup vs baseline: 68.5673x; 1.0247x over previous
import functools

import jax
import jax.numpy as jnp
from jax import lax
from jax.experimental import pallas as pl
from jax.experimental.pallas import tpu as pltpu
from jax.experimental.pallas import tpu_sc as plsc

E = 800000
G = 64
NW = 32
CH = E // NW
CHP = CH + 8
K = 16
INV_TEMP = 20.0
MARGIN = 0.2
PAIR_W = 0.3
NEG_BIG = -1e30
NEG_INF = float("-inf")
TINY = 1.1754943508222875e-38

BR = 64
SUB = 16
ROWS = E // 128
NB = -(-ROWS // BR)


_mesh = plsc.VectorSubcoreMesh(core_axis_name="c", subcore_axis_name="s")

NSTAT = 7
WSTAT = NSTAT * G * K


@functools.partial(
    pl.kernel,
    out_type=jax.ShapeDtypeStruct((NW * WSTAT,), jnp.float32),
    mesh=_mesh,
    compiler_params=pltpu.CompilerParams(needs_layout_passes=False),
    scratch_types=[
        pltpu.VMEM((CHP,), jnp.float32),
        pltpu.VMEM((CHP,), jnp.float32),
        pltpu.VMEM((CHP,), jnp.int32),
        pltpu.VMEM((80,), jnp.int32),
        pltpu.VMEM((WSTAT,), jnp.float32),
    ],
)
def _sc_part(logits_hbm, targets_hbm, eb_hbm, o_st, lg_v, tg_v, eb_v, lb_v, v_st):
    c = lax.axis_index("c")
    s = lax.axis_index("s")
    wid = s * 2 + c
    base = wid * CH
    pltpu.sync_copy(logits_hbm.at[pl.ds(base, CH)], lg_v.at[pl.ds(0, CH)])
    pltpu.sync_copy(targets_hbm.at[pl.ds(base, CH)], tg_v.at[pl.ds(0, CH)])
    pltpu.sync_copy(eb_hbm.at[pl.ds(base, CH)], eb_v.at[pl.ds(0, CH)])

    lane = lax.iota(jnp.int32, 16)

    for r in range(4):
        gvec = lane + r * 16

        def bs_body(i, carry):
            lo, hi = carry
            active = lo < hi
            mid = lax.div(lo + hi, 2)
            vals = plsc.load_gather(eb_v, [mid])
            right = vals < gvec
            lo2 = jnp.where(active & right, mid + 1, lo)
            hi2 = jnp.where(active & (~right), mid, hi)
            return lo2, hi2

        lo, hi = lax.fori_loop(
            0, 15, bs_body,
            (jnp.zeros((16,), jnp.int32), jnp.full((16,), CH, jnp.int32)))
        lb_v[pl.ds(r * 16, 16)] = lo
    lb_v[pl.ds(64, 16)] = jnp.full((16,), CH, jnp.int32)

    def graph_body(g, _):
        gfull = jnp.full((16,), g, jnp.int32)
        start_v = plsc.load_gather(lb_v, [gfull])
        end_v = plsc.load_gather(lb_v, [gfull + 1])
        start = lax.reduce_max(start_v, (0,))
        end = lax.reduce_max(end_v, (0,))
        i0 = lax.div(start, 16)
        i1 = lax.div(end + 15, 16)

        def elem_body(i, carry):
            m_a, s_a, m_p, s_p, pc, top = carry
            off = i * 16
            vraw = lg_v[pl.ds(off, 16)]
            trg = tg_v[pl.ds(off, 16)]
            gidx = off + lane
            valid = (gidx >= start) & (gidx < end)
            posm = valid & (trg > 0.5)
            sc_v = vraw * INV_TEMP
            sv = jnp.where(valid, sc_v, NEG_BIG)
            m_a2 = jnp.maximum(m_a, sv)
            s_a2 = (s_a * jnp.exp(m_a - m_a2)
                    + jnp.where(valid, jnp.exp(sv - m_a2), 0.0))
            pv = jnp.where(posm, sc_v, NEG_BIG)
            m_p2 = jnp.maximum(m_p, pv)
            s_p2 = (s_p * jnp.exp(m_p - m_p2)
                    + jnp.where(posm, jnp.exp(pv - m_p2), 0.0))
            pc2 = pc + jnp.where(posm, 1.0, 0.0)

            negv = jnp.where(valid & (~posm), vraw, NEG_INF)
            kth = lax.reduce_min(top, (0,))

            def merge(t):
                asc = plsc.sort_key_val(negv, negv)[0]
                bit = jnp.maximum(t, asc)
                return plsc.sort_key_val(bit, bit, descending=True)[0]

            top2 = lax.cond(jnp.any(negv > kth), merge, lambda t: t, top)
            return (m_a2, s_a2, m_p2, s_p2, pc2, top2)

        init = (jnp.full((16,), NEG_BIG, jnp.float32),
                jnp.zeros((16,), jnp.float32),
                jnp.full((16,), NEG_BIG, jnp.float32),
                jnp.zeros((16,), jnp.float32),
                jnp.zeros((16,), jnp.float32),
                jnp.full((16,), NEG_INF, jnp.float32))
        m_a, s_a, m_p, s_p, pc, top = lax.fori_loop(i0, i1, elem_body, init)

        slot = g * 16 + lane
        plsc.store_scatter(v_st, [slot + 0 * G * K], m_a)
        plsc.store_scatter(v_st, [slot + 1 * G * K], s_a)
        plsc.store_scatter(v_st, [slot + 2 * G * K], m_p)
        plsc.store_scatter(v_st, [slot + 3 * G * K], s_p)
        plsc.store_scatter(v_st, [slot + 4 * G * K], pc)
        nf = (end - start).astype(jnp.float32)
        cnt_vec = jnp.where(lane == 0, nf, 0.0)
        plsc.store_scatter(v_st, [slot + 5 * G * K], cnt_vec)
        plsc.store_scatter(v_st, [slot + 6 * G * K], top)
        return 0

    lax.fori_loop(0, G, graph_body, 0)

    pltpu.sync_copy(v_st, o_st.at[pl.ds(wid * WSTAT, WSTAT)])


def _softplus(x):
    return jnp.maximum(x, 0.0) + jnp.log(1.0 + jnp.exp(-jnp.abs(x)))


def _tc_body(lg_ref, tg_ref, eb_ref,
             ma_ref, sa_ref, mp_ref, sp_ref, pc_ref, ct_ref, hnc_ref,
             out_ref,
             rep_ref, pos_ref, nv_ref, accum_ref, listw_ref):
    pid = pl.program_id(0)

    @pl.when(pid == 0)
    def _init():
        m_ = ma_ref[...]
        s_ = sa_ref[...]
        M = jnp.max(m_, axis=1, keepdims=True)
        S = jnp.sum(s_ * jnp.exp(m_ - M), axis=1, keepdims=True)
        mp_ = mp_ref[...]
        sp_ = sp_ref[...]
        Mp = jnp.max(mp_, axis=1, keepdims=True)
        Sp = jnp.sum(sp_ * jnp.exp(mp_ - Mp), axis=1, keepdims=True)
        Pos = jnp.sum(pc_ref[...], axis=1, keepdims=True)
        Cnt = jnp.sum(ct_ref[...], axis=1, keepdims=True)
        log_denom = jnp.where(Cnt > 0, M + jnp.log(jnp.maximum(S, TINY)), 0.0)
        log_num = Mp + jnp.log(jnp.maximum(Sp, TINY))
        has_pos = Pos > 0
        log_num_safe = jnp.where(has_pos, log_num, log_denom)
        listwise_sum = jnp.sum(-(log_num_safe - log_denom))
        listwise_den = jnp.maximum(jnp.sum(has_pos.astype(jnp.float32)), 1.0)
        listw_ref[0] = listwise_sum / listwise_den

        cand = hnc_ref[...]
        iota1 = lax.broadcasted_iota(jnp.int32, (G, NW * K), 1)
        nv = jnp.zeros((G, 1), jnp.float32)
        for j in range(K):
            mj = jnp.max(cand, axis=1, keepdims=True)
            first = jnp.min(jnp.where(cand == mj, iota1, NW * K),
                            axis=1, keepdims=True)
            cand = jnp.where(iota1 == first, NEG_INF, cand)
            vj = mj > -1e37
            nv += vj.astype(jnp.float32)
            eff = jnp.where(vj, mj + MARGIN, -1e30)
            rep_ref[:, j] = lax.broadcast_in_dim(eff, (G, SUB, 128), (0, 1))
        pos_ref[...] = Pos
        nv_ref[...] = nv
        accum_ref[...] = jnp.zeros((G, 1), jnp.float32)

    v = lg_ref[...]
    inb = (pid * BR + lax.broadcasted_iota(jnp.int32, (BR, 128), 0)) < ROWS
    posm = (tg_ref[...] > 0.5) & inb
    eb = jnp.where(inb, eb_ref[...], G - 1)
    g_lo = jnp.min(eb)
    g_hi = jnp.max(eb)
    giota = lax.broadcasted_iota(jnp.int32, (G, 1), 0)

    def g_body(g, contrib):
        x3 = rep_ref[g]
        msk = posm & (eb == g)
        ssum = jnp.float32(0.0)
        for t in range(BR // SUB):
            vt = v[t * SUB:(t + 1) * SUB, :]
            v3 = lax.broadcast_in_dim(vt, (K, SUB, 128), (1, 2))
            x = x3 - v3
            e = 1.0 + jnp.exp(-jnp.abs(x))
            p8 = e[0:8] * e[8:16]
            p4 = p8[0:4] * p8[4:8]
            p2 = p4[0:2] * p4[2:4]
            prod = p2[0] * p2[1]
            acc = jnp.sum(jnp.maximum(x, 0.0), axis=0) + jnp.log(prod)
            ssum += jnp.sum(
                jnp.where(msk[t * SUB:(t + 1) * SUB, :], acc, 0.0))
        return contrib + jnp.where(giota == g, ssum, 0.0)

    contrib = lax.fori_loop(g_lo, g_hi + 1, g_body,
                            jnp.zeros((G, 1), jnp.float32))
    accum_ref[...] += contrib

    @pl.when(pid == NB - 1)
    def _fin():
        Pos = pos_ref[...]
        nv = nv_ref[...]
        pair_sum = accum_ref[...]
        pair_cnt = Pos * nv
        cond = (Pos > 0) & (nv > 0)
        mean_g = jnp.where(cond, pair_sum / jnp.maximum(pair_cnt, 1.0), 0.0)
        pgraphs = jnp.sum(cond.astype(jnp.float32))
        pairwise = jnp.sum(mean_g) / jnp.maximum(pgraphs, 1.0)
        out_ref[...] = jnp.full((1, 1), listw_ref[0] + PAIR_W * pairwise,
                                jnp.float32)


_tc_part = pl.pallas_call(
    _tc_body,
    grid=(NB,),
    in_specs=(
        [pl.BlockSpec((BR, 128), lambda i: (i, 0))] * 3
        + [pl.BlockSpec((G, NW * K), lambda i: (0, 0))] * 7
    ),
    out_specs=pl.BlockSpec((1, 1), lambda i: (0, 0)),
    out_shape=jax.ShapeDtypeStruct((1, 1), jnp.float32),
    scratch_shapes=[
        pltpu.VMEM((G, K, SUB, 128), jnp.float32),
        pltpu.VMEM((G, 1), jnp.float32),
        pltpu.VMEM((G, 1), jnp.float32),
        pltpu.VMEM((G, 1), jnp.float32),
        pltpu.SMEM((1,), jnp.float32),
    ],
)


def kernel(logits, targets, edge_batch, num_graphs):
    eb = edge_batch.astype(jnp.int32)
    st = _sc_part(logits, targets, eb).reshape(NW, NSTAT, G, K)
    st = jnp.transpose(st, (1, 2, 0, 3)).reshape(NSTAT, G, NW * K)
    ma, sa, mp, sp, pc, ct, hnc = (st[i] for i in range(NSTAT))

    out = _tc_part(logits.reshape(ROWS, 128), targets.reshape(ROWS, 128),
                   eb.reshape(ROWS, 128), ma, sa, mp, sp, pc, ct, hnc)
    return out.reshape(())


# speedup vs baseline: 73.7149x; 1.0751x over previous
import functools

import jax
import jax.numpy as jnp
from jax import lax
from jax.experimental import pallas as pl
from jax.experimental.pallas import tpu as pltpu
from jax.experimental.pallas import tpu_sc as plsc

E = 800000
G = 64
NW = 32
CH = E // NW
CHP = CH + 8
K = 16
INV_TEMP = 20.0
MARGIN = 0.2
PAIR_W = 0.3
NEG_BIG = -1e30
NEG_INF = float("-inf")
TINY = 1.1754943508222875e-38

BR = 64
SUB = 16
ROWS = E // 128
NB = -(-ROWS // BR)


_mesh = plsc.VectorSubcoreMesh(core_axis_name="c", subcore_axis_name="s")

NSTAT = 7
WSTAT = NSTAT * G * K


@functools.partial(
    pl.kernel,
    out_type=jax.ShapeDtypeStruct((NW * WSTAT,), jnp.float32),
    mesh=_mesh,
    compiler_params=pltpu.CompilerParams(needs_layout_passes=False),
    scratch_types=[
        pltpu.VMEM((CHP,), jnp.float32),
        pltpu.VMEM((CHP,), jnp.float32),
        pltpu.VMEM((CHP,), jnp.int32),
        pltpu.VMEM((80,), jnp.int32),
        pltpu.VMEM((WSTAT,), jnp.float32),
    ],
)
def _sc_part(logits_hbm, targets_hbm, eb_hbm, o_st, lg_v, tg_v, eb_v, lb_v, v_st):
    c = lax.axis_index("c")
    s = lax.axis_index("s")
    wid = s * 2 + c
    base = wid * CH
    pltpu.sync_copy(logits_hbm.at[pl.ds(base, CH)], lg_v.at[pl.ds(0, CH)])
    pltpu.sync_copy(targets_hbm.at[pl.ds(base, CH)], tg_v.at[pl.ds(0, CH)])
    pltpu.sync_copy(eb_hbm.at[pl.ds(base, CH)], eb_v.at[pl.ds(0, CH)])

    lane = lax.iota(jnp.int32, 16)

    for r in range(4):
        gvec = lane + r * 16

        def bs_body(i, carry):
            lo, hi = carry
            active = lo < hi
            mid = lax.div(lo + hi, 2)
            vals = plsc.load_gather(eb_v, [mid])
            right = vals < gvec
            lo2 = jnp.where(active & right, mid + 1, lo)
            hi2 = jnp.where(active & (~right), mid, hi)
            return lo2, hi2

        lo, hi = lax.fori_loop(
            0, 15, bs_body,
            (jnp.zeros((16,), jnp.int32), jnp.full((16,), CH, jnp.int32)))
        lb_v[pl.ds(r * 16, 16)] = lo
    lb_v[pl.ds(64, 16)] = jnp.full((16,), CH, jnp.int32)

    def graph_body(g, _):
        gfull = jnp.full((16,), g, jnp.int32)
        start_v = plsc.load_gather(lb_v, [gfull])
        end_v = plsc.load_gather(lb_v, [gfull + 1])
        start = lax.reduce_max(start_v, (0,))
        end = lax.reduce_max(end_v, (0,))
        i0 = lax.div(start, 16)
        i1 = lax.div(end + 15, 16)

        def elem_body(i, carry):
            m_a, s_a, m_p, s_p, pc, top = carry
            off = i * 16
            vraw = lg_v[pl.ds(off, 16)]
            trg = tg_v[pl.ds(off, 16)]
            gidx = off + lane
            valid = (gidx >= start) & (gidx < end)
            posm = valid & (trg > 0.5)
            sc_v = vraw * INV_TEMP
            sv = jnp.where(valid, sc_v, NEG_BIG)
            m_a2 = jnp.maximum(m_a, sv)
            s_a2 = (s_a * jnp.exp(m_a - m_a2)
                    + jnp.where(valid, jnp.exp(sv - m_a2), 0.0))
            pv = jnp.where(posm, sc_v, NEG_BIG)
            m_p2 = jnp.maximum(m_p, pv)
            s_p2 = (s_p * jnp.exp(m_p - m_p2)
                    + jnp.where(posm, jnp.exp(pv - m_p2), 0.0))
            pc2 = pc + jnp.where(posm, 1.0, 0.0)

            negv = jnp.where(valid & (~posm), vraw, NEG_INF)
            kth = lax.reduce_min(top, (0,))

            def merge(t):
                asc = plsc.sort_key_val(negv, negv)[0]
                bit = jnp.maximum(t, asc)
                return plsc.sort_key_val(bit, bit, descending=True)[0]

            top2 = lax.cond(jnp.any(negv > kth), merge, lambda t: t, top)
            return (m_a2, s_a2, m_p2, s_p2, pc2, top2)

        init = (jnp.full((16,), NEG_BIG, jnp.float32),
                jnp.zeros((16,), jnp.float32),
                jnp.full((16,), NEG_BIG, jnp.float32),
                jnp.zeros((16,), jnp.float32),
                jnp.zeros((16,), jnp.float32),
                jnp.full((16,), NEG_INF, jnp.float32))
        m_a, s_a, m_p, s_p, pc, top = lax.fori_loop(i0, i1, elem_body, init)

        slot = g * 16 + lane
        plsc.store_scatter(v_st, [slot + 0 * G * K], m_a)
        plsc.store_scatter(v_st, [slot + 1 * G * K], s_a)
        plsc.store_scatter(v_st, [slot + 2 * G * K], m_p)
        plsc.store_scatter(v_st, [slot + 3 * G * K], s_p)
        plsc.store_scatter(v_st, [slot + 4 * G * K], pc)
        nf = (end - start).astype(jnp.float32)
        cnt_vec = jnp.where(lane == 0, nf, 0.0)
        plsc.store_scatter(v_st, [slot + 5 * G * K], cnt_vec)
        plsc.store_scatter(v_st, [slot + 6 * G * K], top)
        return 0

    lax.fori_loop(0, G, graph_body, 0)

    pltpu.sync_copy(v_st, o_st.at[pl.ds(wid * WSTAT, WSTAT)])


def _softplus(x):
    return jnp.maximum(x, 0.0) + jnp.log(1.0 + jnp.exp(-jnp.abs(x)))


def _tc_body(lg_ref, tg_ref, eb_ref,
             ma_ref, sa_ref, mp_ref, sp_ref, pc_ref, ct_ref, hnc_ref,
             out_ref,
             rep_ref, pos_ref, nv_ref, accum_ref, listw_ref):
    pid = pl.program_id(0)

    @pl.when(pid == 0)
    def _init():
        m_ = ma_ref[...]
        s_ = sa_ref[...]
        M = jnp.max(m_, axis=1, keepdims=True)
        S = jnp.sum(s_ * jnp.exp(m_ - M), axis=1, keepdims=True)
        mp_ = mp_ref[...]
        sp_ = sp_ref[...]
        Mp = jnp.max(mp_, axis=1, keepdims=True)
        Sp = jnp.sum(sp_ * jnp.exp(mp_ - Mp), axis=1, keepdims=True)
        Pos = jnp.sum(pc_ref[...], axis=1, keepdims=True)
        Cnt = jnp.sum(ct_ref[...], axis=1, keepdims=True)
        log_denom = jnp.where(Cnt > 0, M + jnp.log(jnp.maximum(S, TINY)), 0.0)
        log_num = Mp + jnp.log(jnp.maximum(Sp, TINY))
        has_pos = Pos > 0
        log_num_safe = jnp.where(has_pos, log_num, log_denom)
        listwise_sum = jnp.sum(-(log_num_safe - log_denom))
        listwise_den = jnp.maximum(jnp.sum(has_pos.astype(jnp.float32)), 1.0)
        listw_ref[0] = listwise_sum / listwise_den

        cand = hnc_ref[...]
        iota1 = lax.broadcasted_iota(jnp.int32, (G, NW * K), 1)
        nv = jnp.zeros((G, 1), jnp.float32)
        for j in range(K):
            mj = jnp.max(cand, axis=1, keepdims=True)
            first = jnp.min(jnp.where(cand == mj, iota1, NW * K),
                            axis=1, keepdims=True)
            cand = jnp.where(iota1 == first, NEG_INF, cand)
            vj = mj > -1e37
            nv += vj.astype(jnp.float32)
            eff = jnp.where(vj, jnp.exp(mj + MARGIN), 0.0)
            rep_ref[:, j] = lax.broadcast_in_dim(eff, (G, SUB, 128), (0, 1))
        pos_ref[...] = Pos
        nv_ref[...] = nv
        accum_ref[...] = jnp.zeros((G, 1), jnp.float32)

    v = lg_ref[...]
    inb = (pid * BR + lax.broadcasted_iota(jnp.int32, (BR, 128), 0)) < ROWS
    posm = (tg_ref[...] > 0.5) & inb
    eb = jnp.where(inb, eb_ref[...], G - 1)
    g_lo = jnp.min(eb)
    g_hi = jnp.max(eb)
    giota = lax.broadcasted_iota(jnp.int32, (G, 1), 0)

    negexp = jnp.exp(-v)

    def qprod(eq):
        p = eq[0:2] * eq[2:4]
        return p[0] * p[1]

    def g_body(g, contrib):
        ej = rep_ref[g]
        msk = posm & (eb == g)
        ssum = jnp.float32(0.0)
        for t in range(BR // SUB):
            ft = negexp[t * SUB:(t + 1) * SUB, :]
            f3 = lax.broadcast_in_dim(ft, (K, SUB, 128), (1, 2))
            e = 1.0 + ej * f3
            acc = (jnp.log(qprod(e[0:4])) + jnp.log(qprod(e[4:8]))
                   + jnp.log(qprod(e[8:12])) + jnp.log(qprod(e[12:16])))
            ssum += jnp.sum(
                jnp.where(msk[t * SUB:(t + 1) * SUB, :], acc, 0.0))
        return contrib + jnp.where(giota == g, ssum, 0.0)

    contrib = lax.fori_loop(g_lo, g_hi + 1, g_body,
                            jnp.zeros((G, 1), jnp.float32))
    accum_ref[...] += contrib

    @pl.when(pid == NB - 1)
    def _fin():
        Pos = pos_ref[...]
        nv = nv_ref[...]
        pair_sum = accum_ref[...]
        pair_cnt = Pos * nv
        cond = (Pos > 0) & (nv > 0)
        mean_g = jnp.where(cond, pair_sum / jnp.maximum(pair_cnt, 1.0), 0.0)
        pgraphs = jnp.sum(cond.astype(jnp.float32))
        pairwise = jnp.sum(mean_g) / jnp.maximum(pgraphs, 1.0)
        out_ref[...] = jnp.full((1, 1), listw_ref[0] + PAIR_W * pairwise,
                                jnp.float32)


_tc_part = pl.pallas_call(
    _tc_body,
    grid=(NB,),
    in_specs=(
        [pl.BlockSpec((BR, 128), lambda i: (i, 0))] * 3
        + [pl.BlockSpec((G, NW * K), lambda i: (0, 0))] * 7
    ),
    out_specs=pl.BlockSpec((1, 1), lambda i: (0, 0)),
    out_shape=jax.ShapeDtypeStruct((1, 1), jnp.float32),
    scratch_shapes=[
        pltpu.VMEM((G, K, SUB, 128), jnp.float32),
        pltpu.VMEM((G, 1), jnp.float32),
        pltpu.VMEM((G, 1), jnp.float32),
        pltpu.VMEM((G, 1), jnp.float32),
        pltpu.SMEM((1,), jnp.float32),
    ],
)


def kernel(logits, targets, edge_batch, num_graphs):
    eb = edge_batch.astype(jnp.int32)
    st = _sc_part(logits, targets, eb).reshape(NW, NSTAT, G, K)
    st = jnp.transpose(st, (1, 2, 0, 3)).reshape(NSTAT, G, NW * K)
    ma, sa, mp, sp, pc, ct, hnc = (st[i] for i in range(NSTAT))

    out = _tc_part(logits.reshape(ROWS, 128), targets.reshape(ROWS, 128),
                   eb.reshape(ROWS, 128), ma, sa, mp, sp, pc, ct, hnc)
    return out.reshape(())


# speedup vs baseline: 88.1344x; 1.1956x over previous
import functools

import jax
import jax.numpy as jnp
from jax import lax
from jax.experimental import pallas as pl
from jax.experimental.pallas import tpu as pltpu
from jax.experimental.pallas import tpu_sc as plsc

E = 800000
G = 64
NW = 32
CH = E // NW
CHP = CH + 8
K = 16
INV_TEMP = 20.0
MARGIN = 0.2
PAIR_W = 0.3
NEG_BIG = -1e30
NEG_INF = float("-inf")
TINY = 1.1754943508222875e-38

BR = 128
SUB = 16
ROWS = E // 128
NB = -(-ROWS // BR)


_mesh = plsc.VectorSubcoreMesh(core_axis_name="c", subcore_axis_name="s")

NSTAT = 7
WSTAT = NSTAT * G * K


@functools.partial(
    pl.kernel,
    out_type=jax.ShapeDtypeStruct((NW * WSTAT,), jnp.float32),
    mesh=_mesh,
    compiler_params=pltpu.CompilerParams(needs_layout_passes=False),
    scratch_types=[
        pltpu.VMEM((CHP,), jnp.float32),
        pltpu.VMEM((CHP,), jnp.float32),
        pltpu.VMEM((CHP,), jnp.int32),
        pltpu.VMEM((80,), jnp.int32),
        pltpu.VMEM((WSTAT,), jnp.float32),
    ],
)
def _sc_part(logits_hbm, targets_hbm, eb_hbm, o_st, lg_v, tg_v, eb_v, lb_v, v_st):
    c = lax.axis_index("c")
    s = lax.axis_index("s")
    wid = s * 2 + c
    base = wid * CH
    pltpu.sync_copy(logits_hbm.at[pl.ds(base, CH)], lg_v.at[pl.ds(0, CH)])
    pltpu.sync_copy(targets_hbm.at[pl.ds(base, CH)], tg_v.at[pl.ds(0, CH)])
    pltpu.sync_copy(eb_hbm.at[pl.ds(base, CH)], eb_v.at[pl.ds(0, CH)])

    lane = lax.iota(jnp.int32, 16)

    for r in range(4):
        gvec = lane + r * 16

        def bs_body(i, carry):
            lo, hi = carry
            active = lo < hi
            mid = lax.div(lo + hi, 2)
            vals = plsc.load_gather(eb_v, [mid])
            right = vals < gvec
            lo2 = jnp.where(active & right, mid + 1, lo)
            hi2 = jnp.where(active & (~right), mid, hi)
            return lo2, hi2

        lo, hi = lax.fori_loop(
            0, 15, bs_body,
            (jnp.zeros((16,), jnp.int32), jnp.full((16,), CH, jnp.int32)))
        lb_v[pl.ds(r * 16, 16)] = lo
    lb_v[pl.ds(64, 16)] = jnp.full((16,), CH, jnp.int32)

    def graph_body(g, _):
        gfull = jnp.full((16,), g, jnp.int32)
        start_v = plsc.load_gather(lb_v, [gfull])
        end_v = plsc.load_gather(lb_v, [gfull + 1])
        start = lax.reduce_max(start_v, (0,))
        end = lax.reduce_max(end_v, (0,))
        i0 = lax.div(start, 16)
        i1 = lax.div(end + 15, 16)

        def elem_body(i, carry):
            m_a, s_a, m_p, s_p, pc, top = carry
            off = i * 16
            vraw = lg_v[pl.ds(off, 16)]
            trg = tg_v[pl.ds(off, 16)]
            gidx = off + lane
            valid = (gidx >= start) & (gidx < end)
            posm = valid & (trg > 0.5)
            sc_v = vraw * INV_TEMP
            sv = jnp.where(valid, sc_v, NEG_BIG)
            m_a2 = jnp.maximum(m_a, sv)
            s_a2 = (s_a * jnp.exp(m_a - m_a2)
                    + jnp.where(valid, jnp.exp(sv - m_a2), 0.0))
            pv = jnp.where(posm, sc_v, NEG_BIG)
            m_p2 = jnp.maximum(m_p, pv)
            s_p2 = (s_p * jnp.exp(m_p - m_p2)
                    + jnp.where(posm, jnp.exp(pv - m_p2), 0.0))
            pc2 = pc + jnp.where(posm, 1.0, 0.0)

            negv = jnp.where(valid & (~posm), vraw, NEG_INF)
            kth = lax.reduce_min(top, (0,))

            def merge(t):
                asc = plsc.sort_key_val(negv, negv)[0]
                bit = jnp.maximum(t, asc)
                return plsc.sort_key_val(bit, bit, descending=True)[0]

            top2 = lax.cond(jnp.any(negv > kth), merge, lambda t: t, top)
            return (m_a2, s_a2, m_p2, s_p2, pc2, top2)

        init = (jnp.full((16,), NEG_BIG, jnp.float32),
                jnp.zeros((16,), jnp.float32),
                jnp.full((16,), NEG_BIG, jnp.float32),
                jnp.zeros((16,), jnp.float32),
                jnp.zeros((16,), jnp.float32),
                jnp.full((16,), NEG_INF, jnp.float32))
        m_a, s_a, m_p, s_p, pc, top = lax.fori_loop(i0, i1, elem_body, init)

        slot = g * 16 + lane
        plsc.store_scatter(v_st, [slot + 0 * G * K], m_a)
        plsc.store_scatter(v_st, [slot + 1 * G * K], s_a)
        plsc.store_scatter(v_st, [slot + 2 * G * K], m_p)
        plsc.store_scatter(v_st, [slot + 3 * G * K], s_p)
        plsc.store_scatter(v_st, [slot + 4 * G * K], pc)
        nf = (end - start).astype(jnp.float32)
        cnt_vec = jnp.where(lane == 0, nf, 0.0)
        plsc.store_scatter(v_st, [slot + 5 * G * K], cnt_vec)
        plsc.store_scatter(v_st, [slot + 6 * G * K], top)
        return 0

    lax.fori_loop(0, G, graph_body, 0)

    pltpu.sync_copy(v_st, o_st.at[pl.ds(wid * WSTAT, WSTAT)])


def _softplus(x):
    return jnp.maximum(x, 0.0) + jnp.log(1.0 + jnp.exp(-jnp.abs(x)))


def _tc_body(lg_ref, tg_ref, eb_ref,
             ma_ref, sa_ref, mp_ref, sp_ref, pc_ref, ct_ref, hnc_ref,
             out_ref,
             rep_ref, pos_ref, nv_ref, accum_ref, listw_ref):
    pid = pl.program_id(0)

    @pl.when(pid == 0)
    def _init():
        m_ = ma_ref[...]
        s_ = sa_ref[...]
        M = jnp.max(m_, axis=1, keepdims=True)
        S = jnp.sum(s_ * jnp.exp(m_ - M), axis=1, keepdims=True)
        mp_ = mp_ref[...]
        sp_ = sp_ref[...]
        Mp = jnp.max(mp_, axis=1, keepdims=True)
        Sp = jnp.sum(sp_ * jnp.exp(mp_ - Mp), axis=1, keepdims=True)
        Pos = jnp.sum(pc_ref[...], axis=1, keepdims=True)
        Cnt = jnp.sum(ct_ref[...], axis=1, keepdims=True)
        log_denom = jnp.where(Cnt > 0, M + jnp.log(jnp.maximum(S, TINY)), 0.0)
        log_num = Mp + jnp.log(jnp.maximum(Sp, TINY))
        has_pos = Pos > 0
        log_num_safe = jnp.where(has_pos, log_num, log_denom)
        listwise_sum = jnp.sum(-(log_num_safe - log_denom))
        listwise_den = jnp.maximum(jnp.sum(has_pos.astype(jnp.float32)), 1.0)
        listw_ref[0] = listwise_sum / listwise_den

        cand = hnc_ref[...]
        iota1 = lax.broadcasted_iota(jnp.int32, (G, NW * K), 1)
        nv = jnp.zeros((G, 1), jnp.float32)
        for j in range(K):
            mj = jnp.max(cand, axis=1, keepdims=True)
            first = jnp.min(jnp.where(cand == mj, iota1, NW * K),
                            axis=1, keepdims=True)
            cand = jnp.where(iota1 == first, NEG_INF, cand)
            vj = mj > -1e37
            nv += vj.astype(jnp.float32)
            eff = jnp.where(vj, jnp.exp(mj + MARGIN), 0.0)
            rep_ref[:, j] = lax.broadcast_in_dim(eff, (G, SUB, 128), (0, 1))
        pos_ref[...] = Pos
        nv_ref[...] = nv
        accum_ref[...] = jnp.zeros((G, 1), jnp.float32)

    v = lg_ref[...]
    inb = (pid * BR + lax.broadcasted_iota(jnp.int32, (BR, 128), 0)) < ROWS
    posm = (tg_ref[...] > 0.5) & inb
    eb = jnp.where(inb, eb_ref[...], G - 1)
    g_lo = jnp.min(eb)
    g_hi = jnp.max(eb)
    giota = lax.broadcasted_iota(jnp.int32, (G, 1), 0)

    negexp = jnp.exp(-v)

    def qprod(eq):
        p = eq[0:2] * eq[2:4]
        return p[0] * p[1]

    def g_body(g, contrib):
        ej = rep_ref[g]
        msk = posm & (eb == g)
        ssum = jnp.float32(0.0)
        for t in range(BR // SUB):
            ft = negexp[t * SUB:(t + 1) * SUB, :]
            f3 = lax.broadcast_in_dim(ft, (K, SUB, 128), (1, 2))
            e = 1.0 + ej * f3
            acc = (jnp.log(qprod(e[0:4])) + jnp.log(qprod(e[4:8]))
                   + jnp.log(qprod(e[8:12])) + jnp.log(qprod(e[12:16])))
            ssum += jnp.sum(
                jnp.where(msk[t * SUB:(t + 1) * SUB, :], acc, 0.0))
        return contrib + jnp.where(giota == g, ssum, 0.0)

    contrib = lax.fori_loop(g_lo, g_hi + 1, g_body,
                            jnp.zeros((G, 1), jnp.float32))
    accum_ref[...] += contrib

    @pl.when(pid == NB - 1)
    def _fin():
        Pos = pos_ref[...]
        nv = nv_ref[...]
        pair_sum = accum_ref[...]
        pair_cnt = Pos * nv
        cond = (Pos > 0) & (nv > 0)
        mean_g = jnp.where(cond, pair_sum / jnp.maximum(pair_cnt, 1.0), 0.0)
        pgraphs = jnp.sum(cond.astype(jnp.float32))
        pairwise = jnp.sum(mean_g) / jnp.maximum(pgraphs, 1.0)
        out_ref[...] = jnp.full((1, 1), listw_ref[0] + PAIR_W * pairwise,
                                jnp.float32)


_tc_part = pl.pallas_call(
    _tc_body,
    grid=(NB,),
    in_specs=(
        [pl.BlockSpec((BR, 128), lambda i: (i, 0))] * 3
        + [pl.BlockSpec((G, NW * K), lambda i: (0, 0))] * 7
    ),
    out_specs=pl.BlockSpec((1, 1), lambda i: (0, 0)),
    out_shape=jax.ShapeDtypeStruct((1, 1), jnp.float32),
    scratch_shapes=[
        pltpu.VMEM((G, K, SUB, 128), jnp.float32),
        pltpu.VMEM((G, 1), jnp.float32),
        pltpu.VMEM((G, 1), jnp.float32),
        pltpu.VMEM((G, 1), jnp.float32),
        pltpu.SMEM((1,), jnp.float32),
    ],
)


def kernel(logits, targets, edge_batch, num_graphs):
    eb = edge_batch.astype(jnp.int32)
    st = _sc_part(logits, targets, eb).reshape(NW, NSTAT, G, K)
    st = jnp.transpose(st, (1, 2, 0, 3)).reshape(NSTAT, G, NW * K)
    ma, sa, mp, sp, pc, ct, hnc = (st[i] for i in range(NSTAT))

    out = _tc_part(logits.reshape(ROWS, 128), targets.reshape(ROWS, 128),
                   eb.reshape(ROWS, 128), ma, sa, mp, sp, pc, ct, hnc)
    return out.reshape(())


# speedup vs baseline: 90.1364x; 1.0227x over previous
import functools

import jax
import jax.numpy as jnp
from jax import lax
from jax.experimental import pallas as pl
from jax.experimental.pallas import tpu as pltpu
from jax.experimental.pallas import tpu_sc as plsc

E = 800000
G = 64
NW = 32
CH = E // NW
CHP = CH + 8
K = 16
INV_TEMP = 20.0
MARGIN = 0.2
PAIR_W = 0.3
NEG_BIG = -1e30
NEG_INF = float("-inf")
TINY = 1.1754943508222875e-38

BR = 256
SUB = 16
ROWS = E // 128
NB = -(-ROWS // BR)


_mesh = plsc.VectorSubcoreMesh(core_axis_name="c", subcore_axis_name="s")

NSTAT = 7
WSTAT = NSTAT * G * K


@functools.partial(
    pl.kernel,
    out_type=jax.ShapeDtypeStruct((NW * WSTAT,), jnp.float32),
    mesh=_mesh,
    compiler_params=pltpu.CompilerParams(needs_layout_passes=False),
    scratch_types=[
        pltpu.VMEM((CHP,), jnp.float32),
        pltpu.VMEM((CHP,), jnp.float32),
        pltpu.VMEM((CHP,), jnp.int32),
        pltpu.VMEM((80,), jnp.int32),
        pltpu.VMEM((WSTAT,), jnp.float32),
    ],
)
def _sc_part(logits_hbm, targets_hbm, eb_hbm, o_st, lg_v, tg_v, eb_v, lb_v, v_st):
    c = lax.axis_index("c")
    s = lax.axis_index("s")
    wid = s * 2 + c
    base = wid * CH
    pltpu.sync_copy(logits_hbm.at[pl.ds(base, CH)], lg_v.at[pl.ds(0, CH)])
    pltpu.sync_copy(targets_hbm.at[pl.ds(base, CH)], tg_v.at[pl.ds(0, CH)])
    pltpu.sync_copy(eb_hbm.at[pl.ds(base, CH)], eb_v.at[pl.ds(0, CH)])

    lane = lax.iota(jnp.int32, 16)

    for r in range(4):
        gvec = lane + r * 16

        def bs_body(i, carry):
            lo, hi = carry
            active = lo < hi
            mid = lax.div(lo + hi, 2)
            vals = plsc.load_gather(eb_v, [mid])
            right = vals < gvec
            lo2 = jnp.where(active & right, mid + 1, lo)
            hi2 = jnp.where(active & (~right), mid, hi)
            return lo2, hi2

        lo, hi = lax.fori_loop(
            0, 15, bs_body,
            (jnp.zeros((16,), jnp.int32), jnp.full((16,), CH, jnp.int32)))
        lb_v[pl.ds(r * 16, 16)] = lo
    lb_v[pl.ds(64, 16)] = jnp.full((16,), CH, jnp.int32)

    def graph_body(g, _):
        gfull = jnp.full((16,), g, jnp.int32)
        start_v = plsc.load_gather(lb_v, [gfull])
        end_v = plsc.load_gather(lb_v, [gfull + 1])
        start = lax.reduce_max(start_v, (0,))
        end = lax.reduce_max(end_v, (0,))
        i0 = lax.div(start, 16)
        i1 = lax.div(end + 15, 16)

        def elem_body(i, carry):
            m_a, s_a, m_p, s_p, pc, top = carry
            off = i * 16
            vraw = lg_v[pl.ds(off, 16)]
            trg = tg_v[pl.ds(off, 16)]
            gidx = off + lane
            valid = (gidx >= start) & (gidx < end)
            posm = valid & (trg > 0.5)
            sc_v = vraw * INV_TEMP
            sv = jnp.where(valid, sc_v, NEG_BIG)
            m_a2 = jnp.maximum(m_a, sv)
            s_a2 = (s_a * jnp.exp(m_a - m_a2)
                    + jnp.where(valid, jnp.exp(sv - m_a2), 0.0))
            pv = jnp.where(posm, sc_v, NEG_BIG)
            m_p2 = jnp.maximum(m_p, pv)
            s_p2 = (s_p * jnp.exp(m_p - m_p2)
                    + jnp.where(posm, jnp.exp(pv - m_p2), 0.0))
            pc2 = pc + jnp.where(posm, 1.0, 0.0)

            negv = jnp.where(valid & (~posm), vraw, NEG_INF)
            kth = lax.reduce_min(top, (0,))

            def merge(t):
                asc = plsc.sort_key_val(negv, negv)[0]
                bit = jnp.maximum(t, asc)
                return plsc.sort_key_val(bit, bit, descending=True)[0]

            top2 = lax.cond(jnp.any(negv > kth), merge, lambda t: t, top)
            return (m_a2, s_a2, m_p2, s_p2, pc2, top2)

        init = (jnp.full((16,), NEG_BIG, jnp.float32),
                jnp.zeros((16,), jnp.float32),
                jnp.full((16,), NEG_BIG, jnp.float32),
                jnp.zeros((16,), jnp.float32),
                jnp.zeros((16,), jnp.float32),
                jnp.full((16,), NEG_INF, jnp.float32))
        m_a, s_a, m_p, s_p, pc, top = lax.fori_loop(i0, i1, elem_body, init)

        slot = g * 16 + lane
        plsc.store_scatter(v_st, [slot + 0 * G * K], m_a)
        plsc.store_scatter(v_st, [slot + 1 * G * K], s_a)
        plsc.store_scatter(v_st, [slot + 2 * G * K], m_p)
        plsc.store_scatter(v_st, [slot + 3 * G * K], s_p)
        plsc.store_scatter(v_st, [slot + 4 * G * K], pc)
        nf = (end - start).astype(jnp.float32)
        cnt_vec = jnp.where(lane == 0, nf, 0.0)
        plsc.store_scatter(v_st, [slot + 5 * G * K], cnt_vec)
        plsc.store_scatter(v_st, [slot + 6 * G * K], top)
        return 0

    lax.fori_loop(0, G, graph_body, 0)

    pltpu.sync_copy(v_st, o_st.at[pl.ds(wid * WSTAT, WSTAT)])


def _softplus(x):
    return jnp.maximum(x, 0.0) + jnp.log(1.0 + jnp.exp(-jnp.abs(x)))


def _tc_body(lg_ref, tg_ref, eb_ref,
             ma_ref, sa_ref, mp_ref, sp_ref, pc_ref, ct_ref, hnc_ref,
             out_ref,
             rep_ref, pos_ref, nv_ref, accum_ref, listw_ref):
    pid = pl.program_id(0)

    @pl.when(pid == 0)
    def _init():
        m_ = ma_ref[...]
        s_ = sa_ref[...]
        M = jnp.max(m_, axis=1, keepdims=True)
        S = jnp.sum(s_ * jnp.exp(m_ - M), axis=1, keepdims=True)
        mp_ = mp_ref[...]
        sp_ = sp_ref[...]
        Mp = jnp.max(mp_, axis=1, keepdims=True)
        Sp = jnp.sum(sp_ * jnp.exp(mp_ - Mp), axis=1, keepdims=True)
        Pos = jnp.sum(pc_ref[...], axis=1, keepdims=True)
        Cnt = jnp.sum(ct_ref[...], axis=1, keepdims=True)
        log_denom = jnp.where(Cnt > 0, M + jnp.log(jnp.maximum(S, TINY)), 0.0)
        log_num = Mp + jnp.log(jnp.maximum(Sp, TINY))
        has_pos = Pos > 0
        log_num_safe = jnp.where(has_pos, log_num, log_denom)
        listwise_sum = jnp.sum(-(log_num_safe - log_denom))
        listwise_den = jnp.maximum(jnp.sum(has_pos.astype(jnp.float32)), 1.0)
        listw_ref[0] = listwise_sum / listwise_den

        cand = hnc_ref[...]
        iota1 = lax.broadcasted_iota(jnp.int32, (G, NW * K), 1)
        nv = jnp.zeros((G, 1), jnp.float32)
        for j in range(K):
            mj = jnp.max(cand, axis=1, keepdims=True)
            first = jnp.min(jnp.where(cand == mj, iota1, NW * K),
                            axis=1, keepdims=True)
            cand = jnp.where(iota1 == first, NEG_INF, cand)
            vj = mj > -1e37
            nv += vj.astype(jnp.float32)
            eff = jnp.where(vj, jnp.exp(mj + MARGIN), 0.0)
            rep_ref[:, j] = lax.broadcast_in_dim(eff, (G, SUB, 128), (0, 1))
        pos_ref[...] = Pos
        nv_ref[...] = nv
        accum_ref[...] = jnp.zeros((G, 1), jnp.float32)

    v = lg_ref[...]
    inb = (pid * BR + lax.broadcasted_iota(jnp.int32, (BR, 128), 0)) < ROWS
    posm = (tg_ref[...] > 0.5) & inb
    eb = jnp.where(inb, eb_ref[...], G - 1)
    g_lo = jnp.min(eb)
    g_hi = jnp.max(eb)
    giota = lax.broadcasted_iota(jnp.int32, (G, 1), 0)

    negexp = jnp.exp(-v)

    def qprod(eq):
        p = eq[0:2] * eq[2:4]
        return p[0] * p[1]

    def g_body(g, contrib):
        ej = rep_ref[g]
        msk = posm & (eb == g)
        ssum = jnp.float32(0.0)
        for t in range(BR // SUB):
            ft = negexp[t * SUB:(t + 1) * SUB, :]
            f3 = lax.broadcast_in_dim(ft, (K, SUB, 128), (1, 2))
            e = 1.0 + ej * f3
            acc = (jnp.log(qprod(e[0:4])) + jnp.log(qprod(e[4:8]))
                   + jnp.log(qprod(e[8:12])) + jnp.log(qprod(e[12:16])))
            ssum += jnp.sum(
                jnp.where(msk[t * SUB:(t + 1) * SUB, :], acc, 0.0))
        return contrib + jnp.where(giota == g, ssum, 0.0)

    contrib = lax.fori_loop(g_lo, g_hi + 1, g_body,
                            jnp.zeros((G, 1), jnp.float32))
    accum_ref[...] += contrib

    @pl.when(pid == NB - 1)
    def _fin():
        Pos = pos_ref[...]
        nv = nv_ref[...]
        pair_sum = accum_ref[...]
        pair_cnt = Pos * nv
        cond = (Pos > 0) & (nv > 0)
        mean_g = jnp.where(cond, pair_sum / jnp.maximum(pair_cnt, 1.0), 0.0)
        pgraphs = jnp.sum(cond.astype(jnp.float32))
        pairwise = jnp.sum(mean_g) / jnp.maximum(pgraphs, 1.0)
        out_ref[...] = jnp.full((1, 1), listw_ref[0] + PAIR_W * pairwise,
                                jnp.float32)


_tc_part = pl.pallas_call(
    _tc_body,
    grid=(NB,),
    in_specs=(
        [pl.BlockSpec((BR, 128), lambda i: (i, 0))] * 3
        + [pl.BlockSpec((G, NW * K), lambda i: (0, 0))] * 7
    ),
    out_specs=pl.BlockSpec((1, 1), lambda i: (0, 0)),
    out_shape=jax.ShapeDtypeStruct((1, 1), jnp.float32),
    scratch_shapes=[
        pltpu.VMEM((G, K, SUB, 128), jnp.float32),
        pltpu.VMEM((G, 1), jnp.float32),
        pltpu.VMEM((G, 1), jnp.float32),
        pltpu.VMEM((G, 1), jnp.float32),
        pltpu.SMEM((1,), jnp.float32),
    ],
)


def kernel(logits, targets, edge_batch, num_graphs):
    eb = edge_batch.astype(jnp.int32)
    st = _sc_part(logits, targets, eb).reshape(NW, NSTAT, G, K)
    st = jnp.transpose(st, (1, 2, 0, 3)).reshape(NSTAT, G, NW * K)
    ma, sa, mp, sp, pc, ct, hnc = (st[i] for i in range(NSTAT))

    out = _tc_part(logits.reshape(ROWS, 128), targets.reshape(ROWS, 128),
                   eb.reshape(ROWS, 128), ma, sa, mp, sp, pc, ct, hnc)
    return out.reshape(())
